# Initial kernel scaffold; baseline (speedup 1.0000x reference)
import jax
import jax.numpy as jnp
from jax.experimental import pallas as pl


def kernel(x, h, edges, W_e1, b_e1, W_e2, b_e2, W_m, b_m, W_h1, b_h1, W_h2, b_h2, W_x1, b_x1, W_x2):
    raise NotImplementedError("write your pallas kernel here")


# speedup vs baseline: 3.7513x; 3.7513x over previous
import jax
import jax.numpy as jnp
from jax import lax
from jax.experimental import pallas as pl
from jax.experimental.pallas import tpu as pltpu
from jax.experimental.pallas import tpu_sc as plsc

C = 128
TW = 144
NC = 2
NS = 16
NW = NC * NS
L = 16


def _psi(z):
    return jnp.sign(z) * jnp.log(jnp.abs(z) + 1.0)


def _pick_block(n, cap):
    best = 8
    for b in range(8, cap + 1, 8):
        if n % b == 0:
            best = b
    return best


def _tables_body(h_ref, x_ref, w1a_ref, w1b_ref, ti_ref, tj_ref):
    hb = h_ref[...]
    a = jnp.dot(hb, w1a_ref[...], preferred_element_type=jnp.float32)
    b = jnp.dot(hb, w1b_ref[...], preferred_element_type=jnp.float32)
    xb = x_ref[...]
    z4 = jnp.zeros_like(xb)
    z8 = jnp.zeros((xb.shape[0], 8), jnp.float32)
    ti_ref[...] = jnp.concatenate([a, xb, z4, z8], axis=1)
    tj_ref[...] = jnp.concatenate([b, z4, xb, z8], axis=1)


def _build_tables(h, x, w1a, w1b):
    n = h.shape[0]
    nb = _pick_block(n, 1024)
    grid = (n // nb,)
    return pl.pallas_call(
        _tables_body,
        grid=grid,
        in_specs=[
            pl.BlockSpec((nb, C), lambda i: (i, 0)),
            pl.BlockSpec((nb, 4), lambda i: (i, 0)),
            pl.BlockSpec((C, C), lambda i: (0, 0)),
            pl.BlockSpec((C, C), lambda i: (0, 0)),
        ],
        out_specs=[
            pl.BlockSpec((nb, TW), lambda i: (i, 0)),
            pl.BlockSpec((nb, TW), lambda i: (i, 0)),
        ],
        out_shape=[
            jax.ShapeDtypeStruct((n, TW), jnp.float32),
            jax.ShapeDtypeStruct((n, TW), jnp.float32),
        ],
    )(h, x, w1a, w1b)


def _make_gather(e, chunk, nchunk):
    epw = e // NW
    ngr = chunk // L

    def body(ti_hbm, tj_hbm, i_hbm, j_hbm, gs_hbm, npx_hbm,
             ii_v, jj_v, a_v, b_v, o_v, npx_v, sem_a, sem_b):
        cc = lax.axis_index("c")
        ss = lax.axis_index("s")
        wid = ss * NC + cc
        base0 = wid * epw
        iota = lax.iota(jnp.int32, L)
        zv = jnp.zeros((L,), jnp.float32)
        for g in range(ngr):
            npx_v[6, pl.ds(g * L, L)] = zv
            npx_v[7, pl.ds(g * L, L)] = zv

        def do_chunk(ci, _):
            base = base0 + ci * chunk
            pltpu.sync_copy(i_hbm.at[pl.ds(base, chunk)], ii_v)
            pltpu.sync_copy(j_hbm.at[pl.ds(base, chunk)], jj_v)
            cpa = pltpu.async_copy(ti_hbm.at[ii_v], a_v, sem_a)
            cpb = pltpu.async_copy(tj_hbm.at[jj_v], b_v, sem_b)
            cpa.wait()
            cpb.wait()

            def row(ei, _):
                for k in range(C // L):
                    sl = pl.ds(k * L, L)
                    o_v[ei, sl] = a_v[ei, sl] + b_v[ei, sl]
                return 0

            lax.fori_loop(0, chunk, row, 0)

            for g in range(ngr):
                rows = iota + g * L
                xi = [plsc.load_gather(a_v, [rows, jnp.full((L,), C + c2,
                                                           jnp.int32)])
                      for c2 in range(4)]
                xj = [plsc.load_gather(b_v, [rows, jnp.full((L,), C + 4 + c2,
                                                           jnp.int32)])
                      for c2 in range(4)]
                d = [xi[c2] - xj[c2] for c2 in range(4)]
                nr = d[0] * d[0] - d[1] * d[1] - d[2] * d[2] - d[3] * d[3]
                pr = (xi[0] * xj[0] - xi[1] * xj[1] - xi[2] * xj[2]
                      - xi[3] * xj[3])
                sl = pl.ds(g * L, L)
                npx_v[0, sl] = nr
                npx_v[1, sl] = pr
                for c2 in range(4):
                    npx_v[2 + c2, sl] = xj[c2]

            pltpu.sync_copy(o_v, gs_hbm.at[pl.ds(base, chunk)])
            pltpu.sync_copy(npx_v, npx_hbm.at[:, pl.ds(base, chunk)])
            return 0

        lax.fori_loop(0, nchunk, do_chunk, 0)

    mesh = plsc.VectorSubcoreMesh(core_axis_name="c", subcore_axis_name="s",
                                  num_cores=NC, num_subcores=NS)
    return pl.kernel(
        body,
        out_type=[
            jax.ShapeDtypeStruct((e, C), jnp.float32),
            jax.ShapeDtypeStruct((8, e), jnp.float32),
        ],
        mesh=mesh,
        compiler_params=pltpu.CompilerParams(use_tc_tiling_on_sc=False,
                                             needs_layout_passes=False),
        scratch_types=[
            pltpu.VMEM((chunk,), jnp.int32),
            pltpu.VMEM((chunk,), jnp.int32),
            pltpu.VMEM((chunk, TW), jnp.float32),
            pltpu.VMEM((chunk, TW), jnp.float32),
            pltpu.VMEM((chunk, C), jnp.float32),
            pltpu.VMEM((8, chunk), jnp.float32),
            pltpu.SemaphoreType.DMA,
            pltpu.SemaphoreType.DMA,
        ],
    )


def _edge_body(gs_ref, npx_ref, wn_ref, wp_ref, be1_ref, we2_ref, be2_ref,
               wm_ref, bm_ref, wx1_ref, bx1_ref, wx2_ref, u_ref, v_ref):
    s = gs_ref[...]
    npx = npx_ref[...]
    nr = _psi(npx[0:1, :]).T
    pr = _psi(npx[1:2, :]).T
    pre = s + nr * wn_ref[...] + pr * wp_ref[...] + be1_ref[...]
    m1 = jnp.maximum(pre, 0.0)
    m2 = jnp.maximum(
        jnp.dot(m1, we2_ref[...], preferred_element_type=jnp.float32)
        + be2_ref[...], 0.0)
    att = jax.nn.sigmoid(
        jnp.dot(m2, wm_ref[...], preferred_element_type=jnp.float32)
        + bm_ref[...])
    u_ref[...] = att * m2
    t = jnp.maximum(
        jnp.dot(m2, wx1_ref[...], preferred_element_type=jnp.float32)
        + bx1_ref[...], 0.0)
    px = jnp.dot(t, wx2_ref[...], preferred_element_type=jnp.float32)
    v_ref[...] = px.T * npx[2:6, :]


def _edge_mlp(gs, npx, wn, wp, be1, we2, be2, wm, bm, wx1, bx1, wx2):
    e = gs.shape[0]
    be = _pick_block(e, 2560)
    grid = (e // be,)
    full = lambda shp: pl.BlockSpec(shp, lambda i: (0,) * len(shp))
    return pl.pallas_call(
        _edge_body,
        grid=grid,
        in_specs=[
            pl.BlockSpec((be, C), lambda i: (i, 0)),
            pl.BlockSpec((8, be), lambda i: (0, i)),
            full((1, C)), full((1, C)), full((1, C)),
            full((C, C)), full((1, C)),
            full((C, 1)), full((1, 1)),
            full((C, C)), full((1, C)), full((C, 1)),
        ],
        out_specs=[
            pl.BlockSpec((be, C), lambda i: (i, 0)),
            pl.BlockSpec((4, be), lambda i: (0, i)),
        ],
        out_shape=[
            jax.ShapeDtypeStruct((e, C), jnp.float32),
            jax.ShapeDtypeStruct((4, e), jnp.float32),
        ],
    )(gs, npx, wn, wp, be1, we2, be2, wm, bm, wx1, bx1, wx2)


def _make_scatter(e, n, chunk, nchunk):
    epw = e // NW
    rows_pt = n // NS
    ngr = chunk // L

    def body(u_hbm, v4_hbm, i_hbm, z128_hbm, z8_hbm, wm2_hbm, sg2_hbm,
             idx_v, u_v, v4_v, v_v, wm_sh, sg_sh):
        cc = lax.axis_index("c")
        ss = lax.axis_index("s")
        wid = ss * NC + cc
        rowbase = ss * rows_pt
        iota = lax.iota(jnp.int32, L)
        pltpu.sync_copy(z128_hbm, wm_sh.at[pl.ds(rowbase, rows_pt)])
        pltpu.sync_copy(z8_hbm, sg_sh.at[pl.ds(rowbase, rows_pt)])
        ones = jnp.ones((L,), jnp.float32)
        zv = jnp.zeros((L,), jnp.float32)
        for g in range(ngr):
            rows = iota + g * L
            plsc.store_scatter(v_v, [rows, jnp.full((L,), 4, jnp.int32)], ones)
            for c2 in (5, 6, 7):
                plsc.store_scatter(v_v, [rows, jnp.full((L,), c2, jnp.int32)],
                                   zv)
        plsc.subcore_barrier()

        def do_chunk(ci, _):
            base = wid * epw + ci * chunk
            pltpu.sync_copy(i_hbm.at[pl.ds(base, chunk)], idx_v)
            pltpu.sync_copy(u_hbm.at[pl.ds(base, chunk)], u_v)
            pltpu.sync_copy(v4_hbm.at[:, pl.ds(base, chunk)], v4_v)
            for g in range(ngr):
                rows = iota + g * L
                sl = pl.ds(g * L, L)
                for c2 in range(4):
                    plsc.store_scatter(
                        v_v, [rows, jnp.full((L,), c2, jnp.int32)],
                        v4_v[c2, sl])
            pltpu.sync_copy(u_v, wm_sh.at[idx_v], add=True)
            pltpu.sync_copy(v_v, sg_sh.at[idx_v], add=True)
            return 0

        lax.fori_loop(0, nchunk, do_chunk, 0)
        plsc.subcore_barrier()
        pltpu.sync_copy(wm_sh.at[pl.ds(rowbase, rows_pt)],
                        wm2_hbm.at[cc, pl.ds(rowbase, rows_pt)])
        pltpu.sync_copy(sg_sh.at[pl.ds(rowbase, rows_pt)],
                        sg2_hbm.at[cc, pl.ds(rowbase, rows_pt)])

    mesh = plsc.VectorSubcoreMesh(core_axis_name="c", subcore_axis_name="s",
                                  num_cores=NC, num_subcores=NS)
    return pl.kernel(
        body,
        out_type=[
            jax.ShapeDtypeStruct((NC, n, C), jnp.float32),
            jax.ShapeDtypeStruct((NC, n, 8), jnp.float32),
        ],
        mesh=mesh,
        compiler_params=pltpu.CompilerParams(use_tc_tiling_on_sc=False,
                                             needs_layout_passes=False),
        scratch_types=[
            pltpu.VMEM((chunk,), jnp.int32),
            pltpu.VMEM((chunk, C), jnp.float32),
            pltpu.VMEM((4, chunk), jnp.float32),
            pltpu.VMEM((chunk, 8), jnp.float32),
            pltpu.VMEM_SHARED((n, C), jnp.float32),
            pltpu.VMEM_SHARED((n, 8), jnp.float32),
        ],
    )


def _node_body(h_ref, x_ref, wm2_ref, sg2_ref, wh1a_ref, wh1b_ref,
               bh1_ref, wh2_ref, bh2_ref, ho_ref, xo_ref):
    hb = h_ref[...]
    wm = wm2_ref[0] + wm2_ref[1]
    t = jnp.maximum(
        jnp.dot(hb, wh1a_ref[...], preferred_element_type=jnp.float32)
        + jnp.dot(wm, wh1b_ref[...], preferred_element_type=jnp.float32)
        + bh1_ref[...], 0.0)
    ho_ref[...] = (hb
                   + jnp.dot(t, wh2_ref[...],
                             preferred_element_type=jnp.float32)
                   + bh2_ref[...])
    sg = sg2_ref[0] + sg2_ref[1]
    cnt = jnp.maximum(sg[:, 4:5], 1.0)
    xo_ref[...] = x_ref[...] + 0.001 * (sg[:, 0:4] / cnt)


def _node_update(h, x, wm2, sg2, wh1a, wh1b, bh1, wh2, bh2):
    n = h.shape[0]
    nb = _pick_block(n, 1024)
    grid = (n // nb,)
    full = lambda shp: pl.BlockSpec(shp, lambda i: (0,) * len(shp))
    return pl.pallas_call(
        _node_body,
        grid=grid,
        in_specs=[
            pl.BlockSpec((nb, C), lambda i: (i, 0)),
            pl.BlockSpec((nb, 4), lambda i: (i, 0)),
            pl.BlockSpec((NC, nb, C), lambda i: (0, i, 0)),
            pl.BlockSpec((NC, nb, 8), lambda i: (0, i, 0)),
            full((C, C)), full((C, C)), full((1, C)),
            full((C, C)), full((1, C)),
        ],
        out_specs=[
            pl.BlockSpec((nb, C), lambda i: (i, 0)),
            pl.BlockSpec((nb, 4), lambda i: (i, 0)),
        ],
        out_shape=[
            jax.ShapeDtypeStruct((n, C), jnp.float32),
            jax.ShapeDtypeStruct((n, 4), jnp.float32),
        ],
    )(h, x, wm2, sg2, wh1a, wh1b, bh1, wh2, bh2)


def kernel(x, h, edges, W_e1, b_e1, W_e2, b_e2, W_m, b_m,
           W_h1, b_h1, W_h2, b_h2, W_x1, b_x1, W_x2):
    n = h.shape[0]
    e = edges.shape[1]
    epw = e // NW
    chunk = _pick_block(epw, 128)
    nchunk = epw // chunk

    i32 = edges[0].astype(jnp.int32)
    j32 = edges[1].astype(jnp.int32)

    w1a = W_e1[:C]
    w1b = W_e1[C:2 * C]
    wn = W_e1[2 * C:2 * C + 1]
    wp = W_e1[2 * C + 1:2 * C + 2]

    ti, tj = _build_tables(h, x, w1a, w1b)
    gs, npx = _make_gather(e, chunk, nchunk)(ti, tj, i32, j32)
    u, v4 = _edge_mlp(gs, npx, wn, wp, b_e1.reshape(1, C), W_e2,
                      b_e2.reshape(1, C), W_m, b_m.reshape(1, 1),
                      W_x1, b_x1.reshape(1, C), W_x2)
    z128 = jnp.zeros((n // NS, C), jnp.float32)
    z8 = jnp.zeros((n // NS, 8), jnp.float32)
    wm2, sg2 = _make_scatter(e, n, chunk, nchunk)(u, v4, i32, z128, z8)
    h_out, x_out = _node_update(
        h, x, wm2, sg2, W_h1[:C], W_h1[C:], b_h1.reshape(1, C),
        W_h2, b_h2.reshape(1, C))
    return (h_out, x_out)


# speedup vs baseline: 4.8849x; 1.3022x over previous
import jax
import jax.numpy as jnp
from jax import lax
from jax.experimental import pallas as pl
from jax.experimental.pallas import tpu as pltpu
from jax.experimental.pallas import tpu_sc as plsc

C = 128
TW = 144
NC = 2
NS = 16
NW = NC * NS
L = 16


def _psi(z):
    return jnp.sign(z) * jnp.log(jnp.abs(z) + 1.0)


def _pick_block(n, cap):
    best = 8
    for b in range(8, cap + 1, 8):
        if n % b == 0:
            best = b
    return best


def _tables_body(h_ref, x_ref, w1a_ref, w1b_ref, ti_ref, tj_ref):
    hb = h_ref[...]
    a = jnp.dot(hb, w1a_ref[...], preferred_element_type=jnp.float32)
    b = jnp.dot(hb, w1b_ref[...], preferred_element_type=jnp.float32)
    xb = x_ref[...]
    z4 = jnp.zeros_like(xb)
    z8 = jnp.zeros((xb.shape[0], 8), jnp.float32)
    ti_ref[...] = jnp.concatenate([a, xb, z4, z8], axis=1)
    tj_ref[...] = jnp.concatenate([b, z4, xb, z8], axis=1)


def _build_tables(h, x, w1a, w1b):
    n = h.shape[0]
    nb = _pick_block(n, 1024)
    grid = (n // nb,)
    return pl.pallas_call(
        _tables_body,
        grid=grid,
        in_specs=[
            pl.BlockSpec((nb, C), lambda i: (i, 0)),
            pl.BlockSpec((nb, 4), lambda i: (i, 0)),
            pl.BlockSpec((C, C), lambda i: (0, 0)),
            pl.BlockSpec((C, C), lambda i: (0, 0)),
        ],
        out_specs=[
            pl.BlockSpec((nb, TW), lambda i: (i, 0)),
            pl.BlockSpec((nb, TW), lambda i: (i, 0)),
        ],
        out_shape=[
            jax.ShapeDtypeStruct((n, TW), jnp.float32),
            jax.ShapeDtypeStruct((n, TW), jnp.float32),
        ],
    )(h, x, w1a, w1b)


def _make_gather(e, chunk, nchunk):
    epw = e // NW
    ngr = chunk // L
    last = nchunk - 1
    npairs = (nchunk + 1) // 2

    def body(ti_hbm, tj_hbm, i_hbm, j_hbm, gs_hbm, npx_hbm,
             ii0, jj0, a0, b0, o0, npx0, ii1, jj1, a1, b1, o1, npx1,
             si0, sj0, sa0, sb0, sg0, sn0, si1, sj1, sa1, sb1, sg1, sn1):
        cc = lax.axis_index("c")
        ss = lax.axis_index("s")
        wid = ss * NC + cc
        base0 = wid * epw
        iota = lax.iota(jnp.int32, L)
        zv = jnp.zeros((L,), jnp.float32)
        bufs = [
            (ii0, jj0, a0, b0, o0, npx0, si0, sj0, sa0, sb0, sg0, sn0),
            (ii1, jj1, a1, b1, o1, npx1, si1, sj1, sa1, sb1, sg1, sn1),
        ]
        for _, _, _, _, _, npx_v, *_ in bufs:
            for g in range(ngr):
                npx_v[6, pl.ds(g * L, L)] = zv
                npx_v[7, pl.ds(g * L, L)] = zv

        def issue_idx(k, p):
            ii_v, jj_v = bufs[p][0], bufs[p][1]
            base = base0 + k * chunk
            pltpu.async_copy(i_hbm.at[pl.ds(base, chunk)], ii_v, bufs[p][6])
            pltpu.async_copy(j_hbm.at[pl.ds(base, chunk)], jj_v, bufs[p][7])

        def wait_idx(p):
            pltpu.make_async_copy(i_hbm.at[pl.ds(0, chunk)], bufs[p][0],
                                  bufs[p][6]).wait()
            pltpu.make_async_copy(j_hbm.at[pl.ds(0, chunk)], bufs[p][1],
                                  bufs[p][7]).wait()

        def issue_gather(p):
            pltpu.async_copy(ti_hbm.at[bufs[p][0]], bufs[p][2], bufs[p][8])
            pltpu.async_copy(tj_hbm.at[bufs[p][1]], bufs[p][3], bufs[p][9])

        def wait_gather(p):
            pltpu.make_async_copy(ti_hbm.at[bufs[p][0]], bufs[p][2],
                                  bufs[p][8]).wait()
            pltpu.make_async_copy(tj_hbm.at[bufs[p][1]], bufs[p][3],
                                  bufs[p][9]).wait()

        def issue_out(k, p):
            base = base0 + k * chunk
            pltpu.async_copy(bufs[p][4], gs_hbm.at[pl.ds(base, chunk)],
                             bufs[p][10])
            pltpu.async_copy(bufs[p][5], npx_hbm.at[:, pl.ds(base, chunk)],
                             bufs[p][11])

        def wait_out(p):
            pltpu.make_async_copy(bufs[p][4], gs_hbm.at[pl.ds(0, chunk)],
                                  bufs[p][10]).wait()
            pltpu.make_async_copy(bufs[p][5], npx_hbm.at[:, pl.ds(0, chunk)],
                                  bufs[p][11]).wait()

        def compute(p):
            _, _, a_v, b_v, o_v, npx_v, *_ = bufs[p]

            def row(ei, _):
                for k in range(C // L):
                    sl = pl.ds(k * L, L)
                    o_v[ei, sl] = a_v[ei, sl] + b_v[ei, sl]
                return 0

            lax.fori_loop(0, chunk, row, 0)
            for g in range(ngr):
                rows = iota + g * L
                xi = [plsc.load_gather(a_v, [rows, jnp.full((L,), C + c2,
                                                            jnp.int32)])
                      for c2 in range(4)]
                xj = [plsc.load_gather(b_v, [rows, jnp.full((L,), C + 4 + c2,
                                                            jnp.int32)])
                      for c2 in range(4)]
                d = [xi[c2] - xj[c2] for c2 in range(4)]
                nr = d[0] * d[0] - d[1] * d[1] - d[2] * d[2] - d[3] * d[3]
                pr = (xi[0] * xj[0] - xi[1] * xj[1] - xi[2] * xj[2]
                      - xi[3] * xj[3])
                sl = pl.ds(g * L, L)
                npx_v[0, sl] = nr
                npx_v[1, sl] = pr
                for c2 in range(4):
                    npx_v[2 + c2, sl] = xj[c2]

        def handle(k, p):
            wait_gather(p)

            @pl.when(k + 2 <= last)
            def _():
                issue_idx(k + 2, p)

            @pl.when(k + 1 <= last)
            def _():
                wait_idx(1 - p)
                issue_gather(1 - p)

            @pl.when(k >= 2)
            def _():
                wait_out(p)

            compute(p)
            issue_out(k, p)

        issue_idx(0, 0)
        issue_idx(1, 1)
        wait_idx(0)
        issue_gather(0)

        def pair(m, _):
            handle(2 * m, 0)

            @pl.when(2 * m + 1 <= last)
            def _():
                handle(2 * m + 1, 1)

            return 0

        lax.fori_loop(0, npairs, pair, 0)
        wait_out(0)
        wait_out(1)

    mesh = plsc.VectorSubcoreMesh(core_axis_name="c", subcore_axis_name="s",
                                  num_cores=NC, num_subcores=NS)
    return pl.kernel(
        body,
        out_type=[
            jax.ShapeDtypeStruct((e, C), jnp.float32),
            jax.ShapeDtypeStruct((8, e), jnp.float32),
        ],
        mesh=mesh,
        compiler_params=pltpu.CompilerParams(use_tc_tiling_on_sc=False,
                                             needs_layout_passes=False),
        scratch_types=(
            [
                pltpu.VMEM((chunk,), jnp.int32),
                pltpu.VMEM((chunk,), jnp.int32),
                pltpu.VMEM((chunk, TW), jnp.float32),
                pltpu.VMEM((chunk, TW), jnp.float32),
                pltpu.VMEM((chunk, C), jnp.float32),
                pltpu.VMEM((8, chunk), jnp.float32),
            ] * 2
            + [pltpu.SemaphoreType.DMA] * 12
        ),
    )


def _edge_body(gs_ref, npx_ref, wn_ref, wp_ref, be1_ref, we2_ref, be2_ref,
               wm_ref, bm_ref, wx1_ref, bx1_ref, wx2_ref, u_ref, v_ref):
    s = gs_ref[...]
    npx = npx_ref[...]
    nr = _psi(npx[0:1, :]).T
    pr = _psi(npx[1:2, :]).T
    pre = s + nr * wn_ref[...] + pr * wp_ref[...] + be1_ref[...]
    m1 = jnp.maximum(pre, 0.0)
    m2 = jnp.maximum(
        jnp.dot(m1, we2_ref[...], preferred_element_type=jnp.float32)
        + be2_ref[...], 0.0)
    att = jax.nn.sigmoid(
        jnp.dot(m2, wm_ref[...], preferred_element_type=jnp.float32)
        + bm_ref[...])
    u_ref[...] = att * m2
    t = jnp.maximum(
        jnp.dot(m2, wx1_ref[...], preferred_element_type=jnp.float32)
        + bx1_ref[...], 0.0)
    px = jnp.dot(t, wx2_ref[...], preferred_element_type=jnp.float32)
    v_ref[...] = px.T * npx[2:6, :]


def _edge_mlp(gs, npx, wn, wp, be1, we2, be2, wm, bm, wx1, bx1, wx2):
    e = gs.shape[0]
    be = _pick_block(e, 2560)
    grid = (e // be,)
    full = lambda shp: pl.BlockSpec(shp, lambda i: (0,) * len(shp))
    return pl.pallas_call(
        _edge_body,
        grid=grid,
        in_specs=[
            pl.BlockSpec((be, C), lambda i: (i, 0)),
            pl.BlockSpec((8, be), lambda i: (0, i)),
            full((1, C)), full((1, C)), full((1, C)),
            full((C, C)), full((1, C)),
            full((C, 1)), full((1, 1)),
            full((C, C)), full((1, C)), full((C, 1)),
        ],
        out_specs=[
            pl.BlockSpec((be, C), lambda i: (i, 0)),
            pl.BlockSpec((4, be), lambda i: (0, i)),
        ],
        out_shape=[
            jax.ShapeDtypeStruct((e, C), jnp.float32),
            jax.ShapeDtypeStruct((4, e), jnp.float32),
        ],
    )(gs, npx, wn, wp, be1, we2, be2, wm, bm, wx1, bx1, wx2)


def _make_scatter(e, n, chunk, nchunk):
    epw = e // NW
    rows_pt = n // NS
    ngr = chunk // L

    def body(u_hbm, v4_hbm, i_hbm, z128_hbm, z8_hbm, wm2_hbm, sg2_hbm,
             idx_v, u_v, v4_v, v_v, wm_sh, sg_sh):
        cc = lax.axis_index("c")
        ss = lax.axis_index("s")
        wid = ss * NC + cc
        rowbase = ss * rows_pt
        iota = lax.iota(jnp.int32, L)
        pltpu.sync_copy(z128_hbm, wm_sh.at[pl.ds(rowbase, rows_pt)])
        pltpu.sync_copy(z8_hbm, sg_sh.at[pl.ds(rowbase, rows_pt)])
        ones = jnp.ones((L,), jnp.float32)
        zv = jnp.zeros((L,), jnp.float32)
        for g in range(ngr):
            rows = iota + g * L
            plsc.store_scatter(v_v, [rows, jnp.full((L,), 4, jnp.int32)], ones)
            for c2 in (5, 6, 7):
                plsc.store_scatter(v_v, [rows, jnp.full((L,), c2, jnp.int32)],
                                   zv)
        plsc.subcore_barrier()

        def do_chunk(ci, _):
            base = wid * epw + ci * chunk
            pltpu.sync_copy(i_hbm.at[pl.ds(base, chunk)], idx_v)
            pltpu.sync_copy(u_hbm.at[pl.ds(base, chunk)], u_v)
            pltpu.sync_copy(v4_hbm.at[:, pl.ds(base, chunk)], v4_v)
            for g in range(ngr):
                rows = iota + g * L
                sl = pl.ds(g * L, L)
                for c2 in range(4):
                    plsc.store_scatter(
                        v_v, [rows, jnp.full((L,), c2, jnp.int32)],
                        v4_v[c2, sl])
            pltpu.sync_copy(u_v, wm_sh.at[idx_v], add=True)
            pltpu.sync_copy(v_v, sg_sh.at[idx_v], add=True)
            return 0

        lax.fori_loop(0, nchunk, do_chunk, 0)
        plsc.subcore_barrier()
        pltpu.sync_copy(wm_sh.at[pl.ds(rowbase, rows_pt)],
                        wm2_hbm.at[cc, pl.ds(rowbase, rows_pt)])
        pltpu.sync_copy(sg_sh.at[pl.ds(rowbase, rows_pt)],
                        sg2_hbm.at[cc, pl.ds(rowbase, rows_pt)])

    mesh = plsc.VectorSubcoreMesh(core_axis_name="c", subcore_axis_name="s",
                                  num_cores=NC, num_subcores=NS)
    return pl.kernel(
        body,
        out_type=[
            jax.ShapeDtypeStruct((NC, n, C), jnp.float32),
            jax.ShapeDtypeStruct((NC, n, 8), jnp.float32),
        ],
        mesh=mesh,
        compiler_params=pltpu.CompilerParams(use_tc_tiling_on_sc=False,
                                             needs_layout_passes=False),
        scratch_types=[
            pltpu.VMEM((chunk,), jnp.int32),
            pltpu.VMEM((chunk, C), jnp.float32),
            pltpu.VMEM((4, chunk), jnp.float32),
            pltpu.VMEM((chunk, 8), jnp.float32),
            pltpu.VMEM_SHARED((n, C), jnp.float32),
            pltpu.VMEM_SHARED((n, 8), jnp.float32),
        ],
    )


def _node_body(h_ref, x_ref, wm2_ref, sg2_ref, wh1a_ref, wh1b_ref,
               bh1_ref, wh2_ref, bh2_ref, ho_ref, xo_ref):
    hb = h_ref[...]
    wm = wm2_ref[0] + wm2_ref[1]
    t = jnp.maximum(
        jnp.dot(hb, wh1a_ref[...], preferred_element_type=jnp.float32)
        + jnp.dot(wm, wh1b_ref[...], preferred_element_type=jnp.float32)
        + bh1_ref[...], 0.0)
    ho_ref[...] = (hb
                   + jnp.dot(t, wh2_ref[...],
                             preferred_element_type=jnp.float32)
                   + bh2_ref[...])
    sg = sg2_ref[0] + sg2_ref[1]
    cnt = jnp.maximum(sg[:, 4:5], 1.0)
    xo_ref[...] = x_ref[...] + 0.001 * (sg[:, 0:4] / cnt)


def _node_update(h, x, wm2, sg2, wh1a, wh1b, bh1, wh2, bh2):
    n = h.shape[0]
    nb = _pick_block(n, 1024)
    grid = (n // nb,)
    full = lambda shp: pl.BlockSpec(shp, lambda i: (0,) * len(shp))
    return pl.pallas_call(
        _node_body,
        grid=grid,
        in_specs=[
            pl.BlockSpec((nb, C), lambda i: (i, 0)),
            pl.BlockSpec((nb, 4), lambda i: (i, 0)),
            pl.BlockSpec((NC, nb, C), lambda i: (0, i, 0)),
            pl.BlockSpec((NC, nb, 8), lambda i: (0, i, 0)),
            full((C, C)), full((C, C)), full((1, C)),
            full((C, C)), full((1, C)),
        ],
        out_specs=[
            pl.BlockSpec((nb, C), lambda i: (i, 0)),
            pl.BlockSpec((nb, 4), lambda i: (i, 0)),
        ],
        out_shape=[
            jax.ShapeDtypeStruct((n, C), jnp.float32),
            jax.ShapeDtypeStruct((n, 4), jnp.float32),
        ],
    )(h, x, wm2, sg2, wh1a, wh1b, bh1, wh2, bh2)


def kernel(x, h, edges, W_e1, b_e1, W_e2, b_e2, W_m, b_m,
           W_h1, b_h1, W_h2, b_h2, W_x1, b_x1, W_x2):
    n = h.shape[0]
    e = edges.shape[1]
    epw = e // NW
    chunk = _pick_block(epw, 128)
    nchunk = epw // chunk

    i32 = edges[0].astype(jnp.int32)
    j32 = edges[1].astype(jnp.int32)

    w1a = W_e1[:C]
    w1b = W_e1[C:2 * C]
    wn = W_e1[2 * C:2 * C + 1]
    wp = W_e1[2 * C + 1:2 * C + 2]

    ti, tj = _build_tables(h, x, w1a, w1b)
    gs, npx = _make_gather(e, chunk, nchunk)(ti, tj, i32, j32)
    u, v4 = _edge_mlp(gs, npx, wn, wp, b_e1.reshape(1, C), W_e2,
                      b_e2.reshape(1, C), W_m, b_m.reshape(1, 1),
                      W_x1, b_x1.reshape(1, C), W_x2)
    z128 = jnp.zeros((n // NS, C), jnp.float32)
    z8 = jnp.zeros((n // NS, 8), jnp.float32)
    wm2, sg2 = _make_scatter(e, n, chunk, nchunk)(u, v4, i32, z128, z8)
    h_out, x_out = _node_update(
        h, x, wm2, sg2, W_h1[:C], W_h1[C:], b_h1.reshape(1, C),
        W_h2, b_h2.reshape(1, C))
    return (h_out, x_out)


# speedup vs baseline: 6.0476x; 1.2380x over previous
import jax
import jax.numpy as jnp
from jax import lax
from jax.experimental import pallas as pl
from jax.experimental.pallas import tpu as pltpu
from jax.experimental.pallas import tpu_sc as plsc

C = 128
TW = 144
NC = 2
NS = 16
NW = NC * NS
L = 16


def _psi(z):
    return jnp.sign(z) * jnp.log(jnp.abs(z) + 1.0)


def _pick_block(n, cap):
    best = 8
    for b in range(8, cap + 1, 8):
        if n % b == 0:
            best = b
    return best


def _tables_body(h_ref, x_ref, w1a_ref, w1b_ref, ti_ref, tj_ref):
    hb = h_ref[...]
    a = jnp.dot(hb, w1a_ref[...], preferred_element_type=jnp.float32)
    b = jnp.dot(hb, w1b_ref[...], preferred_element_type=jnp.float32)
    xb = x_ref[...]
    z4 = jnp.zeros_like(xb)
    z8 = jnp.zeros((xb.shape[0], 8), jnp.float32)
    ti_ref[...] = jnp.concatenate([a, xb, z4, z8], axis=1)
    tj_ref[...] = jnp.concatenate([b, z4, xb, z8], axis=1)


def _build_tables(h, x, w1a, w1b):
    n = h.shape[0]
    nb = _pick_block(n, 1024)
    grid = (n // nb,)
    return pl.pallas_call(
        _tables_body,
        grid=grid,
        in_specs=[
            pl.BlockSpec((nb, C), lambda i: (i, 0)),
            pl.BlockSpec((nb, 4), lambda i: (i, 0)),
            pl.BlockSpec((C, C), lambda i: (0, 0)),
            pl.BlockSpec((C, C), lambda i: (0, 0)),
        ],
        out_specs=[
            pl.BlockSpec((nb, TW), lambda i: (i, 0)),
            pl.BlockSpec((nb, TW), lambda i: (i, 0)),
        ],
        out_shape=[
            jax.ShapeDtypeStruct((n, TW), jnp.float32),
            jax.ShapeDtypeStruct((n, TW), jnp.float32),
        ],
    )(h, x, w1a, w1b)


def _make_gather(e, chunk, nchunk):
    epw = e // NW
    ngr = chunk // L
    last = nchunk - 1
    npairs = (nchunk + 1) // 2

    def body(ti_hbm, tj_hbm, i_hbm, j_hbm, gs_hbm, npx_hbm,
             ii0, jj0, a0, b0, o0, npx0, ii1, jj1, a1, b1, o1, npx1,
             si0, sj0, sa0, sb0, sg0, sn0, si1, sj1, sa1, sb1, sg1, sn1):
        cc = lax.axis_index("c")
        ss = lax.axis_index("s")
        wid = ss * NC + cc
        base0 = wid * epw
        iota = lax.iota(jnp.int32, L)
        zv = jnp.zeros((L,), jnp.float32)
        bufs = [
            (ii0, jj0, a0, b0, o0, npx0, si0, sj0, sa0, sb0, sg0, sn0),
            (ii1, jj1, a1, b1, o1, npx1, si1, sj1, sa1, sb1, sg1, sn1),
        ]
        for _, _, _, _, _, npx_v, *_ in bufs:
            for g in range(ngr):
                npx_v[6, pl.ds(g * L, L)] = zv
                npx_v[7, pl.ds(g * L, L)] = zv

        def issue_idx(k, p):
            ii_v, jj_v = bufs[p][0], bufs[p][1]
            base = base0 + k * chunk
            pltpu.async_copy(i_hbm.at[pl.ds(base, chunk)], ii_v, bufs[p][6])
            pltpu.async_copy(j_hbm.at[pl.ds(base, chunk)], jj_v, bufs[p][7])

        def wait_idx(p):
            pltpu.make_async_copy(i_hbm.at[pl.ds(0, chunk)], bufs[p][0],
                                  bufs[p][6]).wait()
            pltpu.make_async_copy(j_hbm.at[pl.ds(0, chunk)], bufs[p][1],
                                  bufs[p][7]).wait()

        def issue_gather(p):
            pltpu.async_copy(ti_hbm.at[bufs[p][0]], bufs[p][2], bufs[p][8])
            pltpu.async_copy(tj_hbm.at[bufs[p][1]], bufs[p][3], bufs[p][9])

        def wait_gather(p):
            pltpu.make_async_copy(ti_hbm.at[bufs[p][0]], bufs[p][2],
                                  bufs[p][8]).wait()
            pltpu.make_async_copy(tj_hbm.at[bufs[p][1]], bufs[p][3],
                                  bufs[p][9]).wait()

        def issue_out(k, p):
            base = base0 + k * chunk
            pltpu.async_copy(bufs[p][4], gs_hbm.at[pl.ds(base, chunk)],
                             bufs[p][10])
            pltpu.async_copy(bufs[p][5], npx_hbm.at[:, pl.ds(base, chunk)],
                             bufs[p][11])

        def wait_out(p):
            pltpu.make_async_copy(bufs[p][4], gs_hbm.at[pl.ds(0, chunk)],
                                  bufs[p][10]).wait()
            pltpu.make_async_copy(bufs[p][5], npx_hbm.at[:, pl.ds(0, chunk)],
                                  bufs[p][11]).wait()

        def compute(p):
            _, _, a_v, b_v, o_v, npx_v, *_ = bufs[p]

            def row(ei, _):
                for k in range(C // L):
                    sl = pl.ds(k * L, L)
                    o_v[ei, sl] = a_v[ei, sl] + b_v[ei, sl]
                return 0

            lax.fori_loop(0, chunk, row, 0)
            for g in range(ngr):
                rows = iota + g * L
                xi = [plsc.load_gather(a_v, [rows, jnp.full((L,), C + c2,
                                                            jnp.int32)])
                      for c2 in range(4)]
                xj = [plsc.load_gather(b_v, [rows, jnp.full((L,), C + 4 + c2,
                                                            jnp.int32)])
                      for c2 in range(4)]
                d = [xi[c2] - xj[c2] for c2 in range(4)]
                nr = d[0] * d[0] - d[1] * d[1] - d[2] * d[2] - d[3] * d[3]
                pr = (xi[0] * xj[0] - xi[1] * xj[1] - xi[2] * xj[2]
                      - xi[3] * xj[3])
                sl = pl.ds(g * L, L)
                npx_v[0, sl] = nr
                npx_v[1, sl] = pr
                for c2 in range(4):
                    npx_v[2 + c2, sl] = xj[c2]

        def handle(k, p):
            wait_gather(p)

            @pl.when(k + 2 <= last)
            def _():
                issue_idx(k + 2, p)

            @pl.when(k + 1 <= last)
            def _():
                wait_idx(1 - p)
                issue_gather(1 - p)

            @pl.when(k >= 2)
            def _():
                wait_out(p)

            compute(p)
            issue_out(k, p)

        issue_idx(0, 0)
        issue_idx(1, 1)
        wait_idx(0)
        issue_gather(0)

        def pair(m, _):
            handle(2 * m, 0)

            @pl.when(2 * m + 1 <= last)
            def _():
                handle(2 * m + 1, 1)

            return 0

        lax.fori_loop(0, npairs, pair, 0)
        wait_out(0)
        wait_out(1)

    mesh = plsc.VectorSubcoreMesh(core_axis_name="c", subcore_axis_name="s",
                                  num_cores=NC, num_subcores=NS)
    return pl.kernel(
        body,
        out_type=[
            jax.ShapeDtypeStruct((e, C), jnp.float32),
            jax.ShapeDtypeStruct((8, e), jnp.float32),
        ],
        mesh=mesh,
        compiler_params=pltpu.CompilerParams(use_tc_tiling_on_sc=False,
                                             needs_layout_passes=False),
        scratch_types=(
            [
                pltpu.VMEM((chunk,), jnp.int32),
                pltpu.VMEM((chunk,), jnp.int32),
                pltpu.VMEM((chunk, TW), jnp.float32),
                pltpu.VMEM((chunk, TW), jnp.float32),
                pltpu.VMEM((chunk, C), jnp.float32),
                pltpu.VMEM((8, chunk), jnp.float32),
            ] * 2
            + [pltpu.SemaphoreType.DMA] * 12
        ),
    )


def _edge_body(gs_ref, npx_ref, wn_ref, wp_ref, be1_ref, we2_ref, be2_ref,
               wm_ref, bm_ref, wx1_ref, bx1_ref, wx2_ref, u_ref, v_ref):
    s = gs_ref[...]
    npx = npx_ref[...]
    nr = _psi(npx[0:1, :]).T
    pr = _psi(npx[1:2, :]).T
    pre = s + nr * wn_ref[...] + pr * wp_ref[...] + be1_ref[...]
    m1 = jnp.maximum(pre, 0.0)
    m2 = jnp.maximum(
        jnp.dot(m1, we2_ref[...], preferred_element_type=jnp.float32)
        + be2_ref[...], 0.0)
    att = jax.nn.sigmoid(
        jnp.dot(m2, wm_ref[...], preferred_element_type=jnp.float32)
        + bm_ref[...])
    u_ref[...] = att * m2
    t = jnp.maximum(
        jnp.dot(m2, wx1_ref[...], preferred_element_type=jnp.float32)
        + bx1_ref[...], 0.0)
    px = jnp.dot(t, wx2_ref[...], preferred_element_type=jnp.float32)
    v_ref[...] = px.T * npx[2:6, :]


def _edge_mlp(gs, npx, wn, wp, be1, we2, be2, wm, bm, wx1, bx1, wx2):
    e = gs.shape[0]
    be = _pick_block(e, 2560)
    grid = (e // be,)
    full = lambda shp: pl.BlockSpec(shp, lambda i: (0,) * len(shp))
    return pl.pallas_call(
        _edge_body,
        grid=grid,
        in_specs=[
            pl.BlockSpec((be, C), lambda i: (i, 0)),
            pl.BlockSpec((8, be), lambda i: (0, i)),
            full((1, C)), full((1, C)), full((1, C)),
            full((C, C)), full((1, C)),
            full((C, 1)), full((1, 1)),
            full((C, C)), full((1, C)), full((C, 1)),
        ],
        out_specs=[
            pl.BlockSpec((be, C), lambda i: (i, 0)),
            pl.BlockSpec((4, be), lambda i: (0, i)),
        ],
        out_shape=[
            jax.ShapeDtypeStruct((e, C), jnp.float32),
            jax.ShapeDtypeStruct((4, e), jnp.float32),
        ],
    )(gs, npx, wn, wp, be1, we2, be2, wm, bm, wx1, bx1, wx2)


def _make_scatter(e, n, chunk, nchunk):
    epw = e // NW
    rows_pt = n // NS
    ngr = chunk // L
    last = nchunk - 1
    ntrip = (nchunk + 2) // 3

    def body(u_hbm, v4_hbm, i_hbm, z128_hbm, z8_hbm, wm2_hbm, sg2_hbm,
             idx0, u0, v40, v0, idx1, u1, v41, v1, idx2, u2, v42, v2,
             wm_sh, sg_sh,
             li0, lu0, lv0, su0, sv0, li1, lu1, lv1, su1, sv1,
             li2, lu2, lv2, su2, sv2):
        cc = lax.axis_index("c")
        ss = lax.axis_index("s")
        wid = ss * NC + cc
        rowbase = ss * rows_pt
        iota = lax.iota(jnp.int32, L)
        bufs = [
            (idx0, u0, v40, v0, li0, lu0, lv0, su0, sv0),
            (idx1, u1, v41, v1, li1, lu1, lv1, su1, sv1),
            (idx2, u2, v42, v2, li2, lu2, lv2, su2, sv2),
        ]
        pltpu.sync_copy(z128_hbm, wm_sh.at[pl.ds(rowbase, rows_pt)])
        pltpu.sync_copy(z8_hbm, sg_sh.at[pl.ds(rowbase, rows_pt)])
        ones = jnp.ones((L,), jnp.float32)
        zv = jnp.zeros((L,), jnp.float32)
        for _, _, _, v_v, *_ in bufs:
            for g in range(ngr):
                rows = iota + g * L
                plsc.store_scatter(v_v, [rows, jnp.full((L,), 4, jnp.int32)],
                                   ones)
                for c2 in (5, 6, 7):
                    plsc.store_scatter(v_v,
                                       [rows, jnp.full((L,), c2, jnp.int32)],
                                       zv)
        plsc.subcore_barrier()

        def issue_loads(k, p):
            idx_v, u_v, v4_v, _, li, lu, lv, _, _ = bufs[p]
            base = wid * epw + k * chunk
            pltpu.async_copy(i_hbm.at[pl.ds(base, chunk)], idx_v, li)
            pltpu.async_copy(u_hbm.at[pl.ds(base, chunk)], u_v, lu)
            pltpu.async_copy(v4_hbm.at[:, pl.ds(base, chunk)], v4_v, lv)

        def wait_loads(p):
            idx_v, u_v, v4_v, _, li, lu, lv, _, _ = bufs[p]
            pltpu.make_async_copy(i_hbm.at[pl.ds(0, chunk)], idx_v, li).wait()
            pltpu.make_async_copy(u_hbm.at[pl.ds(0, chunk)], u_v, lu).wait()
            pltpu.make_async_copy(v4_hbm.at[:, pl.ds(0, chunk)], v4_v,
                                  lv).wait()

        def issue_scats(p):
            idx_v, u_v, _, v_v, _, _, _, su, sv = bufs[p]
            pltpu.async_copy(u_v, wm_sh.at[idx_v], su, add=True)
            pltpu.async_copy(v_v, sg_sh.at[idx_v], sv, add=True)

        def wait_scats(p):
            idx_v, u_v, _, v_v, _, _, _, su, sv = bufs[p]
            pltpu.make_async_copy(u_v, wm_sh.at[idx_v], su).wait()
            pltpu.make_async_copy(v_v, sg_sh.at[idx_v], sv).wait()

        def handle(k, p):
            nxt = (p + 1) % 3
            @pl.when(k + 1 <= last)
            def _():
                @pl.when(k >= 2)
                def _():
                    wait_scats(nxt)

                issue_loads(k + 1, nxt)

            wait_loads(p)
            _, _, v4_v, v_v, *_ = bufs[p]
            for g in range(ngr):
                rows = iota + g * L
                sl = pl.ds(g * L, L)
                for c2 in range(4):
                    plsc.store_scatter(
                        v_v, [rows, jnp.full((L,), c2, jnp.int32)],
                        v4_v[c2, sl])
            issue_scats(p)

        issue_loads(0, 0)

        def trip(m, _):
            handle(3 * m, 0)

            @pl.when(3 * m + 1 <= last)
            def _():
                handle(3 * m + 1, 1)

            @pl.when(3 * m + 2 <= last)
            def _():
                handle(3 * m + 2, 2)

            return 0

        lax.fori_loop(0, ntrip, trip, 0)
        wait_scats((last - 1) % 3)
        wait_scats(last % 3)
        plsc.subcore_barrier()
        pltpu.sync_copy(wm_sh.at[pl.ds(rowbase, rows_pt)],
                        wm2_hbm.at[cc, pl.ds(rowbase, rows_pt)])
        pltpu.sync_copy(sg_sh.at[pl.ds(rowbase, rows_pt)],
                        sg2_hbm.at[cc, pl.ds(rowbase, rows_pt)])

    mesh = plsc.VectorSubcoreMesh(core_axis_name="c", subcore_axis_name="s",
                                  num_cores=NC, num_subcores=NS)
    return pl.kernel(
        body,
        out_type=[
            jax.ShapeDtypeStruct((NC, n, C), jnp.float32),
            jax.ShapeDtypeStruct((NC, n, 8), jnp.float32),
        ],
        mesh=mesh,
        compiler_params=pltpu.CompilerParams(use_tc_tiling_on_sc=False,
                                             needs_layout_passes=False),
        scratch_types=(
            [
                pltpu.VMEM((chunk,), jnp.int32),
                pltpu.VMEM((chunk, C), jnp.float32),
                pltpu.VMEM((4, chunk), jnp.float32),
                pltpu.VMEM((chunk, 8), jnp.float32),
            ] * 3
            + [
                pltpu.VMEM_SHARED((n, C), jnp.float32),
                pltpu.VMEM_SHARED((n, 8), jnp.float32),
            ]
            + [pltpu.SemaphoreType.DMA] * 15
        ),
    )


def _node_body(h_ref, x_ref, wm2_ref, sg2_ref, wh1a_ref, wh1b_ref,
               bh1_ref, wh2_ref, bh2_ref, ho_ref, xo_ref):
    hb = h_ref[...]
    wm = wm2_ref[0] + wm2_ref[1]
    t = jnp.maximum(
        jnp.dot(hb, wh1a_ref[...], preferred_element_type=jnp.float32)
        + jnp.dot(wm, wh1b_ref[...], preferred_element_type=jnp.float32)
        + bh1_ref[...], 0.0)
    ho_ref[...] = (hb
                   + jnp.dot(t, wh2_ref[...],
                             preferred_element_type=jnp.float32)
                   + bh2_ref[...])
    sg = sg2_ref[0] + sg2_ref[1]
    cnt = jnp.maximum(sg[:, 4:5], 1.0)
    xo_ref[...] = x_ref[...] + 0.001 * (sg[:, 0:4] / cnt)


def _node_update(h, x, wm2, sg2, wh1a, wh1b, bh1, wh2, bh2):
    n = h.shape[0]
    nb = _pick_block(n, 1024)
    grid = (n // nb,)
    full = lambda shp: pl.BlockSpec(shp, lambda i: (0,) * len(shp))
    return pl.pallas_call(
        _node_body,
        grid=grid,
        in_specs=[
            pl.BlockSpec((nb, C), lambda i: (i, 0)),
            pl.BlockSpec((nb, 4), lambda i: (i, 0)),
            pl.BlockSpec((NC, nb, C), lambda i: (0, i, 0)),
            pl.BlockSpec((NC, nb, 8), lambda i: (0, i, 0)),
            full((C, C)), full((C, C)), full((1, C)),
            full((C, C)), full((1, C)),
        ],
        out_specs=[
            pl.BlockSpec((nb, C), lambda i: (i, 0)),
            pl.BlockSpec((nb, 4), lambda i: (i, 0)),
        ],
        out_shape=[
            jax.ShapeDtypeStruct((n, C), jnp.float32),
            jax.ShapeDtypeStruct((n, 4), jnp.float32),
        ],
    )(h, x, wm2, sg2, wh1a, wh1b, bh1, wh2, bh2)


def kernel(x, h, edges, W_e1, b_e1, W_e2, b_e2, W_m, b_m,
           W_h1, b_h1, W_h2, b_h2, W_x1, b_x1, W_x2):
    n = h.shape[0]
    e = edges.shape[1]
    epw = e // NW
    chunk = _pick_block(epw, 128)
    nchunk = epw // chunk

    i32 = edges[0].astype(jnp.int32)
    j32 = edges[1].astype(jnp.int32)

    w1a = W_e1[:C]
    w1b = W_e1[C:2 * C]
    wn = W_e1[2 * C:2 * C + 1]
    wp = W_e1[2 * C + 1:2 * C + 2]

    ti, tj = _build_tables(h, x, w1a, w1b)
    gs, npx = _make_gather(e, chunk, nchunk)(ti, tj, i32, j32)
    u, v4 = _edge_mlp(gs, npx, wn, wp, b_e1.reshape(1, C), W_e2,
                      b_e2.reshape(1, C), W_m, b_m.reshape(1, 1),
                      W_x1, b_x1.reshape(1, C), W_x2)
    z128 = jnp.zeros((n // NS, C), jnp.float32)
    z8 = jnp.zeros((n // NS, 8), jnp.float32)
    wm2, sg2 = _make_scatter(e, n, chunk, nchunk)(u, v4, i32, z128, z8)
    h_out, x_out = _node_update(
        h, x, wm2, sg2, W_h1[:C], W_h1[C:], b_h1.reshape(1, C),
        W_h2, b_h2.reshape(1, C))
    return (h_out, x_out)


# speedup vs baseline: 8.1819x; 1.3529x over previous
import jax
import jax.numpy as jnp
from jax import lax
from jax.experimental import pallas as pl
from jax.experimental.pallas import tpu as pltpu
from jax.experimental.pallas import tpu_sc as plsc

C = 128
TW = 144
NC = 2
NS = 16
NW = NC * NS
L = 16


def _psi(z):
    return jnp.sign(z) * jnp.log(jnp.abs(z) + 1.0)


def _pick_block(n, cap):
    best = 8
    for b in range(8, cap + 1, 8):
        if n % b == 0:
            best = b
    return best


def _tables_body(h_ref, w1a_ref, w1b_ref, ti_ref, tj_ref):
    hb = h_ref[...]
    ti_ref[...] = jnp.dot(hb, w1a_ref[...], preferred_element_type=jnp.float32)
    tj_ref[...] = jnp.dot(hb, w1b_ref[...], preferred_element_type=jnp.float32)


def _build_tables(h, w1a, w1b):
    n = h.shape[0]
    nb = _pick_block(n, 1024)
    grid = (n // nb,)
    return pl.pallas_call(
        _tables_body,
        grid=grid,
        in_specs=[
            pl.BlockSpec((nb, C), lambda i: (i, 0)),
            pl.BlockSpec((C, C), lambda i: (0, 0)),
            pl.BlockSpec((C, C), lambda i: (0, 0)),
        ],
        out_specs=[
            pl.BlockSpec((nb, C), lambda i: (i, 0)),
            pl.BlockSpec((nb, C), lambda i: (i, 0)),
        ],
        out_shape=[
            jax.ShapeDtypeStruct((n, C), jnp.float32),
            jax.ShapeDtypeStruct((n, C), jnp.float32),
        ],
    )(h, w1a, w1b)


def _make_gather(e, n, chunk, nchunk):
    epw = e // NW
    ngr = chunk // L
    last = nchunk - 1
    npairs = (nchunk + 1) // 2

    def body(ti_hbm, tj_hbm, x_hbm, i_hbm, j_hbm, gs_hbm, npx_hbm,
             ii0, jj0, a0, b0, o0, npx0, ii1, jj1, a1, b1, o1, npx1, x_loc,
             si0, sj0, sa0, sb0, sg0, sn0, si1, sj1, sa1, sb1, sg1, sn1):
        cc = lax.axis_index("c")
        ss = lax.axis_index("s")
        wid = ss * NC + cc
        base0 = wid * epw
        iota = lax.iota(jnp.int32, L)
        zv = jnp.zeros((L,), jnp.float32)
        bufs = [
            (ii0, jj0, a0, b0, o0, npx0, si0, sj0, sa0, sb0, sg0, sn0),
            (ii1, jj1, a1, b1, o1, npx1, si1, sj1, sa1, sb1, sg1, sn1),
        ]
        pltpu.sync_copy(x_hbm, x_loc)
        for _, _, _, _, _, npx_v, *_ in bufs:
            for g in range(ngr):
                npx_v[6, pl.ds(g * L, L)] = zv
                npx_v[7, pl.ds(g * L, L)] = zv

        def issue_idx(k, p):
            ii_v, jj_v = bufs[p][0], bufs[p][1]
            base = base0 + k * chunk
            pltpu.async_copy(i_hbm.at[pl.ds(base, chunk)], ii_v, bufs[p][6])
            pltpu.async_copy(j_hbm.at[pl.ds(base, chunk)], jj_v, bufs[p][7])

        def wait_idx(p):
            pltpu.make_async_copy(i_hbm.at[pl.ds(0, chunk)], bufs[p][0],
                                  bufs[p][6]).wait()
            pltpu.make_async_copy(j_hbm.at[pl.ds(0, chunk)], bufs[p][1],
                                  bufs[p][7]).wait()

        def issue_gather(p):
            pltpu.async_copy(ti_hbm.at[bufs[p][0]], bufs[p][2], bufs[p][8])
            pltpu.async_copy(tj_hbm.at[bufs[p][1]], bufs[p][3], bufs[p][9])

        def wait_gather(p):
            pltpu.make_async_copy(ti_hbm.at[bufs[p][0]], bufs[p][2],
                                  bufs[p][8]).wait()
            pltpu.make_async_copy(tj_hbm.at[bufs[p][1]], bufs[p][3],
                                  bufs[p][9]).wait()

        def issue_out(k, p):
            base = base0 + k * chunk
            pltpu.async_copy(bufs[p][4], gs_hbm.at[pl.ds(base, chunk)],
                             bufs[p][10])
            pltpu.async_copy(bufs[p][5], npx_hbm.at[:, pl.ds(base, chunk)],
                             bufs[p][11])

        def wait_out(p):
            pltpu.make_async_copy(bufs[p][4], gs_hbm.at[pl.ds(0, chunk)],
                                  bufs[p][10]).wait()
            pltpu.make_async_copy(bufs[p][5], npx_hbm.at[:, pl.ds(0, chunk)],
                                  bufs[p][11]).wait()

        def compute(p):
            ii_v, jj_v, a_v, b_v, o_v, npx_v, *_ = bufs[p]

            def row(ei, _):
                for k in range(C // L):
                    sl = pl.ds(k * L, L)
                    o_v[ei, sl] = a_v[ei, sl] + b_v[ei, sl]
                return 0

            lax.fori_loop(0, chunk, row, 0)
            for g in range(ngr):
                sl = pl.ds(g * L, L)
                i16 = ii_v[sl]
                j16 = jj_v[sl]
                xi = [plsc.load_gather(x_loc, [jnp.full((L,), c2, jnp.int32),
                                               i16])
                      for c2 in range(4)]
                xj = [plsc.load_gather(x_loc, [jnp.full((L,), c2, jnp.int32),
                                               j16])
                      for c2 in range(4)]
                d = [xi[c2] - xj[c2] for c2 in range(4)]
                nr = d[0] * d[0] - d[1] * d[1] - d[2] * d[2] - d[3] * d[3]
                pr = (xi[0] * xj[0] - xi[1] * xj[1] - xi[2] * xj[2]
                      - xi[3] * xj[3])
                npx_v[0, sl] = nr
                npx_v[1, sl] = pr
                for c2 in range(4):
                    npx_v[2 + c2, sl] = xj[c2]

        def handle(k, p):
            wait_gather(p)

            @pl.when(k + 1 <= last)
            def _():
                wait_idx(1 - p)
                issue_gather(1 - p)

            @pl.when(k >= 2)
            def _():
                wait_out(p)

            compute(p)

            @pl.when(k + 2 <= last)
            def _():
                issue_idx(k + 2, p)

            issue_out(k, p)

        issue_idx(0, 0)
        issue_idx(1, 1)
        wait_idx(0)
        issue_gather(0)

        def pair(m, _):
            handle(2 * m, 0)

            @pl.when(2 * m + 1 <= last)
            def _():
                handle(2 * m + 1, 1)

            return 0

        lax.fori_loop(0, npairs, pair, 0)
        wait_out(0)
        wait_out(1)

    mesh = plsc.VectorSubcoreMesh(core_axis_name="c", subcore_axis_name="s",
                                  num_cores=NC, num_subcores=NS)
    return pl.kernel(
        body,
        out_type=[
            jax.ShapeDtypeStruct((e, C), jnp.float32),
            jax.ShapeDtypeStruct((8, e), jnp.float32),
        ],
        mesh=mesh,
        compiler_params=pltpu.CompilerParams(use_tc_tiling_on_sc=False,
                                             needs_layout_passes=False),
        scratch_types=(
            [
                pltpu.VMEM((chunk,), jnp.int32),
                pltpu.VMEM((chunk,), jnp.int32),
                pltpu.VMEM((chunk, C), jnp.float32),
                pltpu.VMEM((chunk, C), jnp.float32),
                pltpu.VMEM((chunk, C), jnp.float32),
                pltpu.VMEM((8, chunk), jnp.float32),
            ] * 2
            + [pltpu.VMEM((4, n), jnp.float32)]
            + [pltpu.SemaphoreType.DMA] * 12
        ),
    )


def _edge_body(gs_ref, npx_ref, wn_ref, wp_ref, be1_ref, we2_ref, be2_ref,
               wm_ref, bm_ref, wx1_ref, bx1_ref, wx2_ref, u_ref, v_ref):
    s = gs_ref[...]
    npx = npx_ref[...]
    nr = _psi(npx[0:1, :]).T
    pr = _psi(npx[1:2, :]).T
    pre = s + nr * wn_ref[...] + pr * wp_ref[...] + be1_ref[...]
    m1 = jnp.maximum(pre, 0.0)
    m2 = jnp.maximum(
        jnp.dot(m1, we2_ref[...], preferred_element_type=jnp.float32)
        + be2_ref[...], 0.0)
    att = jax.nn.sigmoid(
        jnp.dot(m2, wm_ref[...], preferred_element_type=jnp.float32)
        + bm_ref[...])
    u_ref[...] = att * m2
    t = jnp.maximum(
        jnp.dot(m2, wx1_ref[...], preferred_element_type=jnp.float32)
        + bx1_ref[...], 0.0)
    px = jnp.dot(t, wx2_ref[...], preferred_element_type=jnp.float32)
    v_ref[...] = px.T * npx[2:6, :]


def _edge_mlp(gs, npx, wn, wp, be1, we2, be2, wm, bm, wx1, bx1, wx2):
    e = gs.shape[0]
    be = _pick_block(e, 2560)
    grid = (e // be,)
    full = lambda shp: pl.BlockSpec(shp, lambda i: (0,) * len(shp))
    return pl.pallas_call(
        _edge_body,
        grid=grid,
        in_specs=[
            pl.BlockSpec((be, C), lambda i: (i, 0)),
            pl.BlockSpec((8, be), lambda i: (0, i)),
            full((1, C)), full((1, C)), full((1, C)),
            full((C, C)), full((1, C)),
            full((C, 1)), full((1, 1)),
            full((C, C)), full((1, C)), full((C, 1)),
        ],
        out_specs=[
            pl.BlockSpec((be, C), lambda i: (i, 0)),
            pl.BlockSpec((4, be), lambda i: (0, i)),
        ],
        out_shape=[
            jax.ShapeDtypeStruct((e, C), jnp.float32),
            jax.ShapeDtypeStruct((4, e), jnp.float32),
        ],
    )(gs, npx, wn, wp, be1, we2, be2, wm, bm, wx1, bx1, wx2)


def _make_scatter(e, n, chunk, nchunk):
    epw = e // NW
    rows_pt = n // NS
    ngr = chunk // L
    last = nchunk - 1
    ntrip = (nchunk + 2) // 3

    def body(u_hbm, v4_hbm, i_hbm, z128_hbm, z8_hbm, wm2_hbm, sg2_hbm,
             idx0, u0, v40, v0, idx1, u1, v41, v1, idx2, u2, v42, v2,
             wm_sh, sg_sh,
             li0, lu0, lv0, su0, sv0, li1, lu1, lv1, su1, sv1,
             li2, lu2, lv2, su2, sv2):
        cc = lax.axis_index("c")
        ss = lax.axis_index("s")
        wid = ss * NC + cc
        rowbase = ss * rows_pt
        iota = lax.iota(jnp.int32, L)
        bufs = [
            (idx0, u0, v40, v0, li0, lu0, lv0, su0, sv0),
            (idx1, u1, v41, v1, li1, lu1, lv1, su1, sv1),
            (idx2, u2, v42, v2, li2, lu2, lv2, su2, sv2),
        ]
        pltpu.sync_copy(z128_hbm, wm_sh.at[pl.ds(rowbase, rows_pt)])
        pltpu.sync_copy(z8_hbm, sg_sh.at[pl.ds(rowbase, rows_pt)])
        ones = jnp.ones((L,), jnp.float32)
        zv = jnp.zeros((L,), jnp.float32)
        for _, _, _, v_v, *_ in bufs:
            for g in range(ngr):
                rows = iota + g * L
                plsc.store_scatter(v_v, [rows, jnp.full((L,), 4, jnp.int32)],
                                   ones)
                for c2 in (5, 6, 7):
                    plsc.store_scatter(v_v,
                                       [rows, jnp.full((L,), c2, jnp.int32)],
                                       zv)
        plsc.subcore_barrier()

        def issue_loads(k, p):
            idx_v, u_v, v4_v, _, li, lu, lv, _, _ = bufs[p]
            base = wid * epw + k * chunk
            pltpu.async_copy(i_hbm.at[pl.ds(base, chunk)], idx_v, li)
            pltpu.async_copy(u_hbm.at[pl.ds(base, chunk)], u_v, lu)
            pltpu.async_copy(v4_hbm.at[:, pl.ds(base, chunk)], v4_v, lv)

        def wait_loads(p):
            idx_v, u_v, v4_v, _, li, lu, lv, _, _ = bufs[p]
            pltpu.make_async_copy(i_hbm.at[pl.ds(0, chunk)], idx_v, li).wait()
            pltpu.make_async_copy(u_hbm.at[pl.ds(0, chunk)], u_v, lu).wait()
            pltpu.make_async_copy(v4_hbm.at[:, pl.ds(0, chunk)], v4_v,
                                  lv).wait()

        def issue_scats(p):
            idx_v, u_v, _, v_v, _, _, _, su, sv = bufs[p]
            pltpu.async_copy(u_v, wm_sh.at[idx_v], su, add=True)
            pltpu.async_copy(v_v, sg_sh.at[idx_v], sv, add=True)

        def wait_scats(p):
            idx_v, u_v, _, v_v, _, _, _, su, sv = bufs[p]
            pltpu.make_async_copy(u_v, wm_sh.at[idx_v], su).wait()
            pltpu.make_async_copy(v_v, sg_sh.at[idx_v], sv).wait()

        def handle(k, p):
            nxt = (p + 1) % 3
            @pl.when(k + 1 <= last)
            def _():
                @pl.when(k >= 2)
                def _():
                    wait_scats(nxt)

                issue_loads(k + 1, nxt)

            wait_loads(p)
            _, _, v4_v, v_v, *_ = bufs[p]
            for g in range(ngr):
                rows = iota + g * L
                sl = pl.ds(g * L, L)
                for c2 in range(4):
                    plsc.store_scatter(
                        v_v, [rows, jnp.full((L,), c2, jnp.int32)],
                        v4_v[c2, sl])
            issue_scats(p)

        issue_loads(0, 0)

        def trip(m, _):
            handle(3 * m, 0)

            @pl.when(3 * m + 1 <= last)
            def _():
                handle(3 * m + 1, 1)

            @pl.when(3 * m + 2 <= last)
            def _():
                handle(3 * m + 2, 2)

            return 0

        lax.fori_loop(0, ntrip, trip, 0)
        wait_scats((last - 1) % 3)
        wait_scats(last % 3)
        plsc.subcore_barrier()
        pltpu.sync_copy(wm_sh.at[pl.ds(rowbase, rows_pt)],
                        wm2_hbm.at[cc, pl.ds(rowbase, rows_pt)])
        pltpu.sync_copy(sg_sh.at[pl.ds(rowbase, rows_pt)],
                        sg2_hbm.at[cc, pl.ds(rowbase, rows_pt)])

    mesh = plsc.VectorSubcoreMesh(core_axis_name="c", subcore_axis_name="s",
                                  num_cores=NC, num_subcores=NS)
    return pl.kernel(
        body,
        out_type=[
            jax.ShapeDtypeStruct((NC, n, C), jnp.float32),
            jax.ShapeDtypeStruct((NC, n, 8), jnp.float32),
        ],
        mesh=mesh,
        compiler_params=pltpu.CompilerParams(use_tc_tiling_on_sc=False,
                                             needs_layout_passes=False),
        scratch_types=(
            [
                pltpu.VMEM((chunk,), jnp.int32),
                pltpu.VMEM((chunk, C), jnp.float32),
                pltpu.VMEM((4, chunk), jnp.float32),
                pltpu.VMEM((chunk, 8), jnp.float32),
            ] * 3
            + [
                pltpu.VMEM_SHARED((n, C), jnp.float32),
                pltpu.VMEM_SHARED((n, 8), jnp.float32),
            ]
            + [pltpu.SemaphoreType.DMA] * 15
        ),
    )


def _node_body(h_ref, x_ref, wm2_ref, sg2_ref, wh1a_ref, wh1b_ref,
               bh1_ref, wh2_ref, bh2_ref, ho_ref, xo_ref):
    hb = h_ref[...]
    wm = wm2_ref[0] + wm2_ref[1]
    t = jnp.maximum(
        jnp.dot(hb, wh1a_ref[...], preferred_element_type=jnp.float32)
        + jnp.dot(wm, wh1b_ref[...], preferred_element_type=jnp.float32)
        + bh1_ref[...], 0.0)
    ho_ref[...] = (hb
                   + jnp.dot(t, wh2_ref[...],
                             preferred_element_type=jnp.float32)
                   + bh2_ref[...])
    sg = sg2_ref[0] + sg2_ref[1]
    cnt = jnp.maximum(sg[:, 4:5], 1.0)
    xo_ref[...] = x_ref[...] + 0.001 * (sg[:, 0:4] / cnt)


def _node_update(h, x, wm2, sg2, wh1a, wh1b, bh1, wh2, bh2):
    n = h.shape[0]
    nb = _pick_block(n, 1024)
    grid = (n // nb,)
    full = lambda shp: pl.BlockSpec(shp, lambda i: (0,) * len(shp))
    return pl.pallas_call(
        _node_body,
        grid=grid,
        in_specs=[
            pl.BlockSpec((nb, C), lambda i: (i, 0)),
            pl.BlockSpec((nb, 4), lambda i: (i, 0)),
            pl.BlockSpec((NC, nb, C), lambda i: (0, i, 0)),
            pl.BlockSpec((NC, nb, 8), lambda i: (0, i, 0)),
            full((C, C)), full((C, C)), full((1, C)),
            full((C, C)), full((1, C)),
        ],
        out_specs=[
            pl.BlockSpec((nb, C), lambda i: (i, 0)),
            pl.BlockSpec((nb, 4), lambda i: (i, 0)),
        ],
        out_shape=[
            jax.ShapeDtypeStruct((n, C), jnp.float32),
            jax.ShapeDtypeStruct((n, 4), jnp.float32),
        ],
    )(h, x, wm2, sg2, wh1a, wh1b, bh1, wh2, bh2)


def kernel(x, h, edges, W_e1, b_e1, W_e2, b_e2, W_m, b_m,
           W_h1, b_h1, W_h2, b_h2, W_x1, b_x1, W_x2):
    n = h.shape[0]
    e = edges.shape[1]
    epw = e // NW
    chunk = _pick_block(epw, 128)
    nchunk = epw // chunk

    i32 = edges[0].astype(jnp.int32)
    j32 = edges[1].astype(jnp.int32)

    w1a = W_e1[:C]
    w1b = W_e1[C:2 * C]
    wn = W_e1[2 * C:2 * C + 1]
    wp = W_e1[2 * C + 1:2 * C + 2]

    ti, tj = _build_tables(h, w1a, w1b)
    xt = x.T
    gs, npx = _make_gather(e, n, chunk, nchunk)(ti, tj, xt, i32, j32)
    u, v4 = _edge_mlp(gs, npx, wn, wp, b_e1.reshape(1, C), W_e2,
                      b_e2.reshape(1, C), W_m, b_m.reshape(1, 1),
                      W_x1, b_x1.reshape(1, C), W_x2)
    z128 = jnp.zeros((n // NS, C), jnp.float32)
    z8 = jnp.zeros((n // NS, 8), jnp.float32)
    wm2, sg2 = _make_scatter(e, n, chunk, nchunk)(u, v4, i32, z128, z8)
    h_out, x_out = _node_update(
        h, x, wm2, sg2, W_h1[:C], W_h1[C:], b_h1.reshape(1, C),
        W_h2, b_h2.reshape(1, C))
    return (h_out, x_out)


# speedup vs baseline: 8.2064x; 1.0030x over previous
import jax
import jax.numpy as jnp
from jax import lax
from jax.experimental import pallas as pl
from jax.experimental.pallas import tpu as pltpu
from jax.experimental.pallas import tpu_sc as plsc

C = 128
TW = 144
NC = 2
NS = 16
NW = NC * NS
L = 16


def _psi(z):
    return jnp.sign(z) * jnp.log(jnp.abs(z) + 1.0)


def _pick_block(n, cap):
    best = 8
    for b in range(8, cap + 1, 8):
        if n % b == 0:
            best = b
    return best


def _tables_body(h_ref, w1a_ref, w1b_ref, ti_ref, tj_ref):
    hb = h_ref[...]
    ti_ref[...] = jnp.dot(hb, w1a_ref[...], preferred_element_type=jnp.float32)
    tj_ref[...] = jnp.dot(hb, w1b_ref[...], preferred_element_type=jnp.float32)


def _build_tables(h, w1a, w1b):
    n = h.shape[0]
    nb = _pick_block(n, 1024)
    grid = (n // nb,)
    return pl.pallas_call(
        _tables_body,
        grid=grid,
        in_specs=[
            pl.BlockSpec((nb, C), lambda i: (i, 0)),
            pl.BlockSpec((C, C), lambda i: (0, 0)),
            pl.BlockSpec((C, C), lambda i: (0, 0)),
        ],
        out_specs=[
            pl.BlockSpec((nb, C), lambda i: (i, 0)),
            pl.BlockSpec((nb, C), lambda i: (i, 0)),
        ],
        out_shape=[
            jax.ShapeDtypeStruct((n, C), jnp.float32),
            jax.ShapeDtypeStruct((n, C), jnp.float32),
        ],
    )(h, w1a, w1b)


def _make_gather(e, n, chunk, nchunk):
    epw = e // NW
    ngr = chunk // L
    last = nchunk - 1
    npairs = (nchunk + 1) // 2

    def body(ti_hbm, tj_hbm, x_hbm, i_hbm, j_hbm, gs_hbm, npx_hbm,
             ii0, jj0, a0, b0, o0, npx0, ii1, jj1, a1, b1, o1, npx1, x_loc,
             si0, sj0, sa0, sb0, sg0, sn0, si1, sj1, sa1, sb1, sg1, sn1):
        cc = lax.axis_index("c")
        ss = lax.axis_index("s")
        wid = ss * NC + cc
        base0 = wid * epw
        iota = lax.iota(jnp.int32, L)
        zv = jnp.zeros((L,), jnp.float32)
        bufs = [
            (ii0, jj0, a0, b0, o0, npx0, si0, sj0, sa0, sb0, sg0, sn0),
            (ii1, jj1, a1, b1, o1, npx1, si1, sj1, sa1, sb1, sg1, sn1),
        ]
        pltpu.sync_copy(x_hbm, x_loc)
        for _, _, _, _, _, npx_v, *_ in bufs:
            for g in range(ngr):
                npx_v[6, pl.ds(g * L, L)] = zv
                npx_v[7, pl.ds(g * L, L)] = zv

        def issue_idx(k, p):
            ii_v, jj_v = bufs[p][0], bufs[p][1]
            base = base0 + k * chunk
            pltpu.async_copy(i_hbm.at[pl.ds(base, chunk)], ii_v, bufs[p][6])
            pltpu.async_copy(j_hbm.at[pl.ds(base, chunk)], jj_v, bufs[p][7])

        def wait_idx(p):
            pltpu.make_async_copy(i_hbm.at[pl.ds(0, chunk)], bufs[p][0],
                                  bufs[p][6]).wait()
            pltpu.make_async_copy(j_hbm.at[pl.ds(0, chunk)], bufs[p][1],
                                  bufs[p][7]).wait()

        def issue_gather(p):
            pltpu.async_copy(ti_hbm.at[bufs[p][0]], bufs[p][2], bufs[p][8])
            pltpu.async_copy(tj_hbm.at[bufs[p][1]], bufs[p][3], bufs[p][9])

        def wait_gather(p):
            pltpu.make_async_copy(ti_hbm.at[bufs[p][0]], bufs[p][2],
                                  bufs[p][8]).wait()
            pltpu.make_async_copy(tj_hbm.at[bufs[p][1]], bufs[p][3],
                                  bufs[p][9]).wait()

        def issue_out(k, p):
            base = base0 + k * chunk
            pltpu.async_copy(bufs[p][4], gs_hbm.at[pl.ds(base, chunk)],
                             bufs[p][10])
            pltpu.async_copy(bufs[p][5], npx_hbm.at[:, pl.ds(base, chunk)],
                             bufs[p][11])

        def wait_out(p):
            pltpu.make_async_copy(bufs[p][4], gs_hbm.at[pl.ds(0, chunk)],
                                  bufs[p][10]).wait()
            pltpu.make_async_copy(bufs[p][5], npx_hbm.at[:, pl.ds(0, chunk)],
                                  bufs[p][11]).wait()

        def compute(p):
            ii_v, jj_v, a_v, b_v, o_v, npx_v, *_ = bufs[p]

            def row(ei, _):
                for k in range(C // L):
                    sl = pl.ds(k * L, L)
                    o_v[ei, sl] = a_v[ei, sl] + b_v[ei, sl]
                return 0

            lax.fori_loop(0, chunk, row, 0)
            for g in range(ngr):
                sl = pl.ds(g * L, L)
                i16 = ii_v[sl]
                j16 = jj_v[sl]
                xi = [plsc.load_gather(x_loc, [jnp.full((L,), c2, jnp.int32),
                                               i16])
                      for c2 in range(4)]
                xj = [plsc.load_gather(x_loc, [jnp.full((L,), c2, jnp.int32),
                                               j16])
                      for c2 in range(4)]
                d = [xi[c2] - xj[c2] for c2 in range(4)]
                nr = d[0] * d[0] - d[1] * d[1] - d[2] * d[2] - d[3] * d[3]
                pr = (xi[0] * xj[0] - xi[1] * xj[1] - xi[2] * xj[2]
                      - xi[3] * xj[3])
                npx_v[0, sl] = nr
                npx_v[1, sl] = pr
                for c2 in range(4):
                    npx_v[2 + c2, sl] = xj[c2]

        def handle(k, p):
            wait_gather(p)

            @pl.when(k + 1 <= last)
            def _():
                wait_idx(1 - p)
                issue_gather(1 - p)

            @pl.when(k >= 2)
            def _():
                wait_out(p)

            compute(p)

            @pl.when(k + 2 <= last)
            def _():
                issue_idx(k + 2, p)

            issue_out(k, p)

        issue_idx(0, 0)
        issue_idx(1, 1)
        wait_idx(0)
        issue_gather(0)

        def pair(m, _):
            handle(2 * m, 0)

            @pl.when(2 * m + 1 <= last)
            def _():
                handle(2 * m + 1, 1)

            return 0

        lax.fori_loop(0, npairs, pair, 0)
        wait_out(0)
        wait_out(1)

    mesh = plsc.VectorSubcoreMesh(core_axis_name="c", subcore_axis_name="s",
                                  num_cores=NC, num_subcores=NS)
    return pl.kernel(
        body,
        out_type=[
            jax.ShapeDtypeStruct((e, C), jnp.float32),
            jax.ShapeDtypeStruct((8, e), jnp.float32),
        ],
        mesh=mesh,
        compiler_params=pltpu.CompilerParams(use_tc_tiling_on_sc=False,
                                             needs_layout_passes=False),
        scratch_types=(
            [
                pltpu.VMEM((chunk,), jnp.int32),
                pltpu.VMEM((chunk,), jnp.int32),
                pltpu.VMEM((chunk, C), jnp.float32),
                pltpu.VMEM((chunk, C), jnp.float32),
                pltpu.VMEM((chunk, C), jnp.float32),
                pltpu.VMEM((8, chunk), jnp.float32),
            ] * 2
            + [pltpu.VMEM((4, n), jnp.float32)]
            + [pltpu.SemaphoreType.DMA] * 12
        ),
    )


def _edge_body(gs_ref, npx_ref, wn_ref, wp_ref, be1_ref, we2_ref, be2_ref,
               wm_ref, bm_ref, wx1_ref, bx1_ref, wx2_ref, u_ref, v_ref):
    s = gs_ref[...]
    npx = npx_ref[...]
    nr = _psi(npx[0:1, :]).T
    pr = _psi(npx[1:2, :]).T
    pre = s + nr * wn_ref[...] + pr * wp_ref[...] + be1_ref[...]
    m1 = jnp.maximum(pre, 0.0)
    m2 = jnp.maximum(
        jnp.dot(m1, we2_ref[...], preferred_element_type=jnp.float32)
        + be2_ref[...], 0.0)
    att = jax.nn.sigmoid(
        jnp.dot(m2, wm_ref[...], preferred_element_type=jnp.float32)
        + bm_ref[...])
    u_ref[...] = att * m2
    t = jnp.maximum(
        jnp.dot(m2, wx1_ref[...], preferred_element_type=jnp.float32)
        + bx1_ref[...], 0.0)
    px = jnp.dot(t, wx2_ref[...], preferred_element_type=jnp.float32)
    v_ref[...] = px.T * npx[2:6, :]


def _edge_mlp(gs, npx, wn, wp, be1, we2, be2, wm, bm, wx1, bx1, wx2):
    e = gs.shape[0]
    be = _pick_block(e, 3200)
    grid = (e // be,)
    full = lambda shp: pl.BlockSpec(shp, lambda i: (0,) * len(shp))
    return pl.pallas_call(
        _edge_body,
        grid=grid,
        in_specs=[
            pl.BlockSpec((be, C), lambda i: (i, 0)),
            pl.BlockSpec((8, be), lambda i: (0, i)),
            full((1, C)), full((1, C)), full((1, C)),
            full((C, C)), full((1, C)),
            full((C, 1)), full((1, 1)),
            full((C, C)), full((1, C)), full((C, 1)),
        ],
        out_specs=[
            pl.BlockSpec((be, C), lambda i: (i, 0)),
            pl.BlockSpec((4, be), lambda i: (0, i)),
        ],
        out_shape=[
            jax.ShapeDtypeStruct((e, C), jnp.float32),
            jax.ShapeDtypeStruct((4, e), jnp.float32),
        ],
    )(gs, npx, wn, wp, be1, we2, be2, wm, bm, wx1, bx1, wx2)


def _make_scatter(e, n, chunk, nchunk):
    epw = e // NW
    rows_pt = n // NS
    ngr = chunk // L
    last = nchunk - 1
    ntrip = (nchunk + 2) // 3

    def body(u_hbm, v4_hbm, i_hbm, z128_hbm, z8_hbm, wm2_hbm, sg2_hbm,
             idx0, u0, v40, v0, idx1, u1, v41, v1, idx2, u2, v42, v2,
             wm_sh, sg_sh,
             li0, lu0, lv0, su0, sv0, li1, lu1, lv1, su1, sv1,
             li2, lu2, lv2, su2, sv2):
        cc = lax.axis_index("c")
        ss = lax.axis_index("s")
        wid = ss * NC + cc
        rowbase = ss * rows_pt
        iota = lax.iota(jnp.int32, L)
        bufs = [
            (idx0, u0, v40, v0, li0, lu0, lv0, su0, sv0),
            (idx1, u1, v41, v1, li1, lu1, lv1, su1, sv1),
            (idx2, u2, v42, v2, li2, lu2, lv2, su2, sv2),
        ]
        pltpu.sync_copy(z128_hbm, wm_sh.at[pl.ds(rowbase, rows_pt)])
        pltpu.sync_copy(z8_hbm, sg_sh.at[pl.ds(rowbase, rows_pt)])
        ones = jnp.ones((L,), jnp.float32)
        zv = jnp.zeros((L,), jnp.float32)
        for _, _, _, v_v, *_ in bufs:
            for g in range(ngr):
                rows = iota + g * L
                plsc.store_scatter(v_v, [rows, jnp.full((L,), 4, jnp.int32)],
                                   ones)
                for c2 in (5, 6, 7):
                    plsc.store_scatter(v_v,
                                       [rows, jnp.full((L,), c2, jnp.int32)],
                                       zv)
        plsc.subcore_barrier()

        def issue_loads(k, p):
            idx_v, u_v, v4_v, _, li, lu, lv, _, _ = bufs[p]
            base = wid * epw + k * chunk
            pltpu.async_copy(i_hbm.at[pl.ds(base, chunk)], idx_v, li)
            pltpu.async_copy(u_hbm.at[pl.ds(base, chunk)], u_v, lu)
            pltpu.async_copy(v4_hbm.at[:, pl.ds(base, chunk)], v4_v, lv)

        def wait_loads(p):
            idx_v, u_v, v4_v, _, li, lu, lv, _, _ = bufs[p]
            pltpu.make_async_copy(i_hbm.at[pl.ds(0, chunk)], idx_v, li).wait()
            pltpu.make_async_copy(u_hbm.at[pl.ds(0, chunk)], u_v, lu).wait()
            pltpu.make_async_copy(v4_hbm.at[:, pl.ds(0, chunk)], v4_v,
                                  lv).wait()

        def issue_scats(p):
            idx_v, u_v, _, v_v, _, _, _, su, sv = bufs[p]
            pltpu.async_copy(u_v, wm_sh.at[idx_v], su, add=True)
            pltpu.async_copy(v_v, sg_sh.at[idx_v], sv, add=True)

        def wait_scats(p):
            idx_v, u_v, _, v_v, _, _, _, su, sv = bufs[p]
            pltpu.make_async_copy(u_v, wm_sh.at[idx_v], su).wait()
            pltpu.make_async_copy(v_v, sg_sh.at[idx_v], sv).wait()

        def handle(k, p):
            nxt = (p + 1) % 3
            @pl.when(k + 1 <= last)
            def _():
                @pl.when(k >= 2)
                def _():
                    wait_scats(nxt)

                issue_loads(k + 1, nxt)

            wait_loads(p)
            _, _, v4_v, v_v, *_ = bufs[p]
            for g in range(ngr):
                rows = iota + g * L
                sl = pl.ds(g * L, L)
                for c2 in range(4):
                    plsc.store_scatter(
                        v_v, [rows, jnp.full((L,), c2, jnp.int32)],
                        v4_v[c2, sl])
            issue_scats(p)

        issue_loads(0, 0)

        def trip(m, _):
            handle(3 * m, 0)

            @pl.when(3 * m + 1 <= last)
            def _():
                handle(3 * m + 1, 1)

            @pl.when(3 * m + 2 <= last)
            def _():
                handle(3 * m + 2, 2)

            return 0

        lax.fori_loop(0, ntrip, trip, 0)
        wait_scats((last - 1) % 3)
        wait_scats(last % 3)
        plsc.subcore_barrier()
        pltpu.sync_copy(wm_sh.at[pl.ds(rowbase, rows_pt)],
                        wm2_hbm.at[cc, pl.ds(rowbase, rows_pt)])
        pltpu.sync_copy(sg_sh.at[pl.ds(rowbase, rows_pt)],
                        sg2_hbm.at[cc, pl.ds(rowbase, rows_pt)])

    mesh = plsc.VectorSubcoreMesh(core_axis_name="c", subcore_axis_name="s",
                                  num_cores=NC, num_subcores=NS)
    return pl.kernel(
        body,
        out_type=[
            jax.ShapeDtypeStruct((NC, n, C), jnp.float32),
            jax.ShapeDtypeStruct((NC, n, 8), jnp.float32),
        ],
        mesh=mesh,
        compiler_params=pltpu.CompilerParams(use_tc_tiling_on_sc=False,
                                             needs_layout_passes=False),
        scratch_types=(
            [
                pltpu.VMEM((chunk,), jnp.int32),
                pltpu.VMEM((chunk, C), jnp.float32),
                pltpu.VMEM((4, chunk), jnp.float32),
                pltpu.VMEM((chunk, 8), jnp.float32),
            ] * 3
            + [
                pltpu.VMEM_SHARED((n, C), jnp.float32),
                pltpu.VMEM_SHARED((n, 8), jnp.float32),
            ]
            + [pltpu.SemaphoreType.DMA] * 15
        ),
    )


def _node_body(h_ref, x_ref, wm2_ref, sg2_ref, wh1a_ref, wh1b_ref,
               bh1_ref, wh2_ref, bh2_ref, ho_ref, xo_ref):
    hb = h_ref[...]
    wm = wm2_ref[0] + wm2_ref[1]
    t = jnp.maximum(
        jnp.dot(hb, wh1a_ref[...], preferred_element_type=jnp.float32)
        + jnp.dot(wm, wh1b_ref[...], preferred_element_type=jnp.float32)
        + bh1_ref[...], 0.0)
    ho_ref[...] = (hb
                   + jnp.dot(t, wh2_ref[...],
                             preferred_element_type=jnp.float32)
                   + bh2_ref[...])
    sg = sg2_ref[0] + sg2_ref[1]
    cnt = jnp.maximum(sg[:, 4:5], 1.0)
    xo_ref[...] = x_ref[...] + 0.001 * (sg[:, 0:4] / cnt)


def _node_update(h, x, wm2, sg2, wh1a, wh1b, bh1, wh2, bh2):
    n = h.shape[0]
    nb = _pick_block(n, 1024)
    grid = (n // nb,)
    full = lambda shp: pl.BlockSpec(shp, lambda i: (0,) * len(shp))
    return pl.pallas_call(
        _node_body,
        grid=grid,
        in_specs=[
            pl.BlockSpec((nb, C), lambda i: (i, 0)),
            pl.BlockSpec((nb, 4), lambda i: (i, 0)),
            pl.BlockSpec((NC, nb, C), lambda i: (0, i, 0)),
            pl.BlockSpec((NC, nb, 8), lambda i: (0, i, 0)),
            full((C, C)), full((C, C)), full((1, C)),
            full((C, C)), full((1, C)),
        ],
        out_specs=[
            pl.BlockSpec((nb, C), lambda i: (i, 0)),
            pl.BlockSpec((nb, 4), lambda i: (i, 0)),
        ],
        out_shape=[
            jax.ShapeDtypeStruct((n, C), jnp.float32),
            jax.ShapeDtypeStruct((n, 4), jnp.float32),
        ],
    )(h, x, wm2, sg2, wh1a, wh1b, bh1, wh2, bh2)


def kernel(x, h, edges, W_e1, b_e1, W_e2, b_e2, W_m, b_m,
           W_h1, b_h1, W_h2, b_h2, W_x1, b_x1, W_x2):
    n = h.shape[0]
    e = edges.shape[1]
    epw = e // NW
    chunk = _pick_block(epw, 128)
    nchunk = epw // chunk

    i32 = edges[0].astype(jnp.int32)
    j32 = edges[1].astype(jnp.int32)

    w1a = W_e1[:C]
    w1b = W_e1[C:2 * C]
    wn = W_e1[2 * C:2 * C + 1]
    wp = W_e1[2 * C + 1:2 * C + 2]

    ti, tj = _build_tables(h, w1a, w1b)
    xt = x.T
    gs, npx = _make_gather(e, n, chunk, nchunk)(ti, tj, xt, i32, j32)
    u, v4 = _edge_mlp(gs, npx, wn, wp, b_e1.reshape(1, C), W_e2,
                      b_e2.reshape(1, C), W_m, b_m.reshape(1, 1),
                      W_x1, b_x1.reshape(1, C), W_x2)
    z128 = jnp.zeros((n // NS, C), jnp.float32)
    z8 = jnp.zeros((n // NS, 8), jnp.float32)
    wm2, sg2 = _make_scatter(e, n, chunk, nchunk)(u, v4, i32, z128, z8)
    h_out, x_out = _node_update(
        h, x, wm2, sg2, W_h1[:C], W_h1[C:], b_h1.reshape(1, C),
        W_h2, b_h2.reshape(1, C))
    return (h_out, x_out)


# speedup vs baseline: 9.5981x; 1.1696x over previous
import jax
import jax.numpy as jnp
from jax import lax
from jax.experimental import pallas as pl
from jax.experimental.pallas import tpu as pltpu
from jax.experimental.pallas import tpu_sc as plsc

C = 128
TW = 144
NC = 2
NS = 16
NW = NC * NS
L = 16


def _psi(z):
    return jnp.sign(z) * jnp.log(jnp.abs(z) + 1.0)


def _pick_block(n, cap):
    best = 8
    for b in range(8, cap + 1, 8):
        if n % b == 0:
            best = b
    return best


def _tables_body(h_ref, w1a_ref, w1b_ref, ti_ref, tj_ref):
    hb = h_ref[...]
    ti_ref[...] = jnp.dot(hb, w1a_ref[...], preferred_element_type=jnp.float32)
    tj_ref[...] = jnp.dot(hb, w1b_ref[...], preferred_element_type=jnp.float32)


def _build_tables(h, w1a, w1b):
    n = h.shape[0]
    nb = _pick_block(n, 1024)
    grid = (n // nb,)
    return pl.pallas_call(
        _tables_body,
        grid=grid,
        in_specs=[
            pl.BlockSpec((nb, C), lambda i: (i, 0)),
            pl.BlockSpec((C, C), lambda i: (0, 0)),
            pl.BlockSpec((C, C), lambda i: (0, 0)),
        ],
        out_specs=[
            pl.BlockSpec((nb, C), lambda i: (i, 0)),
            pl.BlockSpec((nb, C), lambda i: (i, 0)),
        ],
        out_shape=[
            jax.ShapeDtypeStruct((n, C), jnp.float32),
            jax.ShapeDtypeStruct((n, C), jnp.float32),
        ],
    )(h, w1a, w1b)


def _make_gather(e, n, chunk, nchunk):
    epw = e // NW
    ngr = chunk // L
    last = nchunk - 1
    npairs = (nchunk + 1) // 2

    def body(ti_hbm, tj_hbm, x_hbm, i_hbm, j_hbm, gs_hbm, npx_hbm,
             ii0, jj0, a0, b0, o0, npx0, ii1, jj1, a1, b1, o1, npx1, x_loc,
             si0, sj0, sa0, sb0, sg0, sn0, si1, sj1, sa1, sb1, sg1, sn1):
        cc = lax.axis_index("c")
        ss = lax.axis_index("s")
        wid = ss * NC + cc
        base0 = wid * epw
        iota = lax.iota(jnp.int32, L)
        zv = jnp.zeros((L,), jnp.float32)
        bufs = [
            (ii0, jj0, a0, b0, o0, npx0, si0, sj0, sa0, sb0, sg0, sn0),
            (ii1, jj1, a1, b1, o1, npx1, si1, sj1, sa1, sb1, sg1, sn1),
        ]
        pltpu.sync_copy(x_hbm, x_loc)
        for _, _, _, _, _, npx_v, *_ in bufs:
            for g in range(ngr):
                npx_v[6, pl.ds(g * L, L)] = zv
                npx_v[7, pl.ds(g * L, L)] = zv

        def issue_idx(k, p):
            ii_v, jj_v = bufs[p][0], bufs[p][1]
            base = base0 + k * chunk
            pltpu.async_copy(i_hbm.at[pl.ds(base, chunk)], ii_v, bufs[p][6])
            pltpu.async_copy(j_hbm.at[pl.ds(base, chunk)], jj_v, bufs[p][7])

        def wait_idx(p):
            pltpu.make_async_copy(i_hbm.at[pl.ds(0, chunk)], bufs[p][0],
                                  bufs[p][6]).wait()
            pltpu.make_async_copy(j_hbm.at[pl.ds(0, chunk)], bufs[p][1],
                                  bufs[p][7]).wait()

        def issue_gather(p):
            pltpu.async_copy(ti_hbm.at[bufs[p][0]], bufs[p][2], bufs[p][8])
            pltpu.async_copy(tj_hbm.at[bufs[p][1]], bufs[p][3], bufs[p][9])

        def wait_gather(p):
            pltpu.make_async_copy(ti_hbm.at[bufs[p][0]], bufs[p][2],
                                  bufs[p][8]).wait()
            pltpu.make_async_copy(tj_hbm.at[bufs[p][1]], bufs[p][3],
                                  bufs[p][9]).wait()

        def issue_out(k, p):
            base = base0 + k * chunk
            pltpu.async_copy(bufs[p][4], gs_hbm.at[pl.ds(base, chunk)],
                             bufs[p][10])
            pltpu.async_copy(bufs[p][5], npx_hbm.at[:, pl.ds(base, chunk)],
                             bufs[p][11])

        def wait_out(p):
            pltpu.make_async_copy(bufs[p][4], gs_hbm.at[pl.ds(0, chunk)],
                                  bufs[p][10]).wait()
            pltpu.make_async_copy(bufs[p][5], npx_hbm.at[:, pl.ds(0, chunk)],
                                  bufs[p][11]).wait()

        def compute(p):
            ii_v, jj_v, a_v, b_v, o_v, npx_v, *_ = bufs[p]

            def row(ei, _):
                for k in range(C // L):
                    sl = pl.ds(k * L, L)
                    o_v[ei, sl] = a_v[ei, sl] + b_v[ei, sl]
                return 0

            lax.fori_loop(0, chunk, row, 0)
            for g in range(ngr):
                sl = pl.ds(g * L, L)
                i16 = ii_v[sl]
                j16 = jj_v[sl]
                xi = [plsc.load_gather(x_loc, [jnp.full((L,), c2, jnp.int32),
                                               i16])
                      for c2 in range(4)]
                xj = [plsc.load_gather(x_loc, [jnp.full((L,), c2, jnp.int32),
                                               j16])
                      for c2 in range(4)]
                d = [xi[c2] - xj[c2] for c2 in range(4)]
                nr = d[0] * d[0] - d[1] * d[1] - d[2] * d[2] - d[3] * d[3]
                pr = (xi[0] * xj[0] - xi[1] * xj[1] - xi[2] * xj[2]
                      - xi[3] * xj[3])
                npx_v[0, sl] = nr
                npx_v[1, sl] = pr
                for c2 in range(4):
                    npx_v[2 + c2, sl] = xj[c2]

        def handle(k, p):
            wait_gather(p)

            @pl.when(k + 1 <= last)
            def _():
                wait_idx(1 - p)
                issue_gather(1 - p)

            @pl.when(k >= 2)
            def _():
                wait_out(p)

            compute(p)

            @pl.when(k + 2 <= last)
            def _():
                issue_idx(k + 2, p)

            issue_out(k, p)

        issue_idx(0, 0)
        issue_idx(1, 1)
        wait_idx(0)
        issue_gather(0)

        def pair(m, _):
            handle(2 * m, 0)

            @pl.when(2 * m + 1 <= last)
            def _():
                handle(2 * m + 1, 1)

            return 0

        lax.fori_loop(0, npairs, pair, 0)
        wait_out(0)
        wait_out(1)

    mesh = plsc.VectorSubcoreMesh(core_axis_name="c", subcore_axis_name="s",
                                  num_cores=NC, num_subcores=NS)
    return pl.kernel(
        body,
        out_type=[
            jax.ShapeDtypeStruct((e, C), jnp.float32),
            jax.ShapeDtypeStruct((8, e), jnp.float32),
        ],
        mesh=mesh,
        compiler_params=pltpu.CompilerParams(use_tc_tiling_on_sc=False,
                                             needs_layout_passes=False),
        scratch_types=(
            [
                pltpu.VMEM((chunk,), jnp.int32),
                pltpu.VMEM((chunk,), jnp.int32),
                pltpu.VMEM((chunk, C), jnp.float32),
                pltpu.VMEM((chunk, C), jnp.float32),
                pltpu.VMEM((chunk, C), jnp.float32),
                pltpu.VMEM((8, chunk), jnp.float32),
            ] * 2
            + [pltpu.VMEM((4, n), jnp.float32)]
            + [pltpu.SemaphoreType.DMA] * 12
        ),
    )


def _edge_body(gs_ref, npx_ref, wnp_ref, be1_ref, we2_ref, be2_ref,
               wm_ref, bm_ref, wx1_ref, bx1_ref, wx2_ref, u_ref, v_ref):
    s = gs_ref[...]
    npx = npx_ref[...]
    nrp = _psi(npx[0:2, :])
    geo = lax.dot_general(nrp, wnp_ref[...], (((0,), (0,)), ((), ())),
                          preferred_element_type=jnp.float32)
    pre = s + geo + be1_ref[...]
    m1 = jnp.maximum(pre, 0.0)
    m2 = jnp.maximum(
        jnp.dot(m1, we2_ref[...], preferred_element_type=jnp.float32)
        + be2_ref[...], 0.0)
    att = jax.nn.sigmoid(
        jnp.dot(m2, wm_ref[...], preferred_element_type=jnp.float32)
        + bm_ref[...])
    u_ref[...] = att * m2
    t = jnp.maximum(
        jnp.dot(m2, wx1_ref[...], preferred_element_type=jnp.float32)
        + bx1_ref[...], 0.0)
    pxt = lax.dot_general(wx2_ref[...], t, (((0,), (1,)), ((), ())),
                          preferred_element_type=jnp.float32)
    v_ref[...] = pxt * npx[2:6, :]


def _edge_mlp(gs, npx, wnp, be1, we2, be2, wm, bm, wx1, bx1, wx2):
    e = gs.shape[0]
    be = _pick_block(e, 3200)
    grid = (e // be,)
    full = lambda shp: pl.BlockSpec(shp, lambda i: (0,) * len(shp))
    return pl.pallas_call(
        _edge_body,
        grid=grid,
        in_specs=[
            pl.BlockSpec((be, C), lambda i: (i, 0)),
            pl.BlockSpec((8, be), lambda i: (0, i)),
            full((2, C)), full((1, C)),
            full((C, C)), full((1, C)),
            full((C, 1)), full((1, 1)),
            full((C, C)), full((1, C)), full((C, 1)),
        ],
        out_specs=[
            pl.BlockSpec((be, C), lambda i: (i, 0)),
            pl.BlockSpec((4, be), lambda i: (0, i)),
        ],
        out_shape=[
            jax.ShapeDtypeStruct((e, C), jnp.float32),
            jax.ShapeDtypeStruct((4, e), jnp.float32),
        ],
    )(gs, npx, wnp, be1, we2, be2, wm, bm, wx1, bx1, wx2)


def _make_scatter(e, n, chunk, nchunk):
    epw = e // NW
    rows_pt = n // NS
    ngr = chunk // L
    last = nchunk - 1
    ntrip = (nchunk + 2) // 3

    def body(u_hbm, v4_hbm, i_hbm, z128_hbm, z8_hbm, wm2_hbm, sg2_hbm,
             idx0, u0, v40, v0, idx1, u1, v41, v1, idx2, u2, v42, v2,
             wm_sh, sg_sh,
             li0, lu0, lv0, su0, sv0, li1, lu1, lv1, su1, sv1,
             li2, lu2, lv2, su2, sv2):
        cc = lax.axis_index("c")
        ss = lax.axis_index("s")
        wid = ss * NC + cc
        rowbase = ss * rows_pt
        iota = lax.iota(jnp.int32, L)
        bufs = [
            (idx0, u0, v40, v0, li0, lu0, lv0, su0, sv0),
            (idx1, u1, v41, v1, li1, lu1, lv1, su1, sv1),
            (idx2, u2, v42, v2, li2, lu2, lv2, su2, sv2),
        ]
        pltpu.sync_copy(z128_hbm, wm_sh.at[pl.ds(rowbase, rows_pt)])
        pltpu.sync_copy(z8_hbm, sg_sh.at[pl.ds(rowbase, rows_pt)])
        ones = jnp.ones((L,), jnp.float32)
        zv = jnp.zeros((L,), jnp.float32)
        for _, _, _, v_v, *_ in bufs:
            for g in range(ngr):
                rows = iota + g * L
                plsc.store_scatter(v_v, [rows, jnp.full((L,), 4, jnp.int32)],
                                   ones)
                for c2 in (5, 6, 7):
                    plsc.store_scatter(v_v,
                                       [rows, jnp.full((L,), c2, jnp.int32)],
                                       zv)
        plsc.subcore_barrier()

        def issue_loads(k, p):
            idx_v, u_v, v4_v, _, li, lu, lv, _, _ = bufs[p]
            base = wid * epw + k * chunk
            pltpu.async_copy(i_hbm.at[pl.ds(base, chunk)], idx_v, li)
            pltpu.async_copy(u_hbm.at[pl.ds(base, chunk)], u_v, lu)
            pltpu.async_copy(v4_hbm.at[:, pl.ds(base, chunk)], v4_v, lv)

        def wait_loads(p):
            idx_v, u_v, v4_v, _, li, lu, lv, _, _ = bufs[p]
            pltpu.make_async_copy(i_hbm.at[pl.ds(0, chunk)], idx_v, li).wait()
            pltpu.make_async_copy(u_hbm.at[pl.ds(0, chunk)], u_v, lu).wait()
            pltpu.make_async_copy(v4_hbm.at[:, pl.ds(0, chunk)], v4_v,
                                  lv).wait()

        def issue_scats(p):
            idx_v, u_v, _, v_v, _, _, _, su, sv = bufs[p]
            pltpu.async_copy(u_v, wm_sh.at[idx_v], su, add=True)
            pltpu.async_copy(v_v, sg_sh.at[idx_v], sv, add=True)

        def wait_scats(p):
            idx_v, u_v, _, v_v, _, _, _, su, sv = bufs[p]
            pltpu.make_async_copy(u_v, wm_sh.at[idx_v], su).wait()
            pltpu.make_async_copy(v_v, sg_sh.at[idx_v], sv).wait()

        def handle(k, p):
            nxt = (p + 1) % 3
            @pl.when(k + 1 <= last)
            def _():
                @pl.when(k >= 2)
                def _():
                    wait_scats(nxt)

                issue_loads(k + 1, nxt)

            wait_loads(p)
            _, _, v4_v, v_v, *_ = bufs[p]
            for g in range(ngr):
                rows = iota + g * L
                sl = pl.ds(g * L, L)
                for c2 in range(4):
                    plsc.store_scatter(
                        v_v, [rows, jnp.full((L,), c2, jnp.int32)],
                        v4_v[c2, sl])
            issue_scats(p)

        issue_loads(0, 0)

        def trip(m, _):
            handle(3 * m, 0)

            @pl.when(3 * m + 1 <= last)
            def _():
                handle(3 * m + 1, 1)

            @pl.when(3 * m + 2 <= last)
            def _():
                handle(3 * m + 2, 2)

            return 0

        lax.fori_loop(0, ntrip, trip, 0)
        wait_scats((last - 1) % 3)
        wait_scats(last % 3)
        plsc.subcore_barrier()
        pltpu.sync_copy(wm_sh.at[pl.ds(rowbase, rows_pt)],
                        wm2_hbm.at[cc, pl.ds(rowbase, rows_pt)])
        pltpu.sync_copy(sg_sh.at[pl.ds(rowbase, rows_pt)],
                        sg2_hbm.at[cc, pl.ds(rowbase, rows_pt)])

    mesh = plsc.VectorSubcoreMesh(core_axis_name="c", subcore_axis_name="s",
                                  num_cores=NC, num_subcores=NS)
    return pl.kernel(
        body,
        out_type=[
            jax.ShapeDtypeStruct((NC, n, C), jnp.float32),
            jax.ShapeDtypeStruct((NC, n, 8), jnp.float32),
        ],
        mesh=mesh,
        compiler_params=pltpu.CompilerParams(use_tc_tiling_on_sc=False,
                                             needs_layout_passes=False),
        scratch_types=(
            [
                pltpu.VMEM((chunk,), jnp.int32),
                pltpu.VMEM((chunk, C), jnp.float32),
                pltpu.VMEM((4, chunk), jnp.float32),
                pltpu.VMEM((chunk, 8), jnp.float32),
            ] * 3
            + [
                pltpu.VMEM_SHARED((n, C), jnp.float32),
                pltpu.VMEM_SHARED((n, 8), jnp.float32),
            ]
            + [pltpu.SemaphoreType.DMA] * 15
        ),
    )


def _node_body(h_ref, x_ref, wm2_ref, sg2_ref, wh1a_ref, wh1b_ref,
               bh1_ref, wh2_ref, bh2_ref, ho_ref, xo_ref):
    hb = h_ref[...]
    wm = wm2_ref[0] + wm2_ref[1]
    t = jnp.maximum(
        jnp.dot(hb, wh1a_ref[...], preferred_element_type=jnp.float32)
        + jnp.dot(wm, wh1b_ref[...], preferred_element_type=jnp.float32)
        + bh1_ref[...], 0.0)
    ho_ref[...] = (hb
                   + jnp.dot(t, wh2_ref[...],
                             preferred_element_type=jnp.float32)
                   + bh2_ref[...])
    sg = sg2_ref[0] + sg2_ref[1]
    cnt = jnp.maximum(sg[:, 4:5], 1.0)
    xo_ref[...] = x_ref[...] + 0.001 * (sg[:, 0:4] / cnt)


def _node_update(h, x, wm2, sg2, wh1a, wh1b, bh1, wh2, bh2):
    n = h.shape[0]
    nb = _pick_block(n, 1024)
    grid = (n // nb,)
    full = lambda shp: pl.BlockSpec(shp, lambda i: (0,) * len(shp))
    return pl.pallas_call(
        _node_body,
        grid=grid,
        in_specs=[
            pl.BlockSpec((nb, C), lambda i: (i, 0)),
            pl.BlockSpec((nb, 4), lambda i: (i, 0)),
            pl.BlockSpec((NC, nb, C), lambda i: (0, i, 0)),
            pl.BlockSpec((NC, nb, 8), lambda i: (0, i, 0)),
            full((C, C)), full((C, C)), full((1, C)),
            full((C, C)), full((1, C)),
        ],
        out_specs=[
            pl.BlockSpec((nb, C), lambda i: (i, 0)),
            pl.BlockSpec((nb, 4), lambda i: (i, 0)),
        ],
        out_shape=[
            jax.ShapeDtypeStruct((n, C), jnp.float32),
            jax.ShapeDtypeStruct((n, 4), jnp.float32),
        ],
    )(h, x, wm2, sg2, wh1a, wh1b, bh1, wh2, bh2)


def kernel(x, h, edges, W_e1, b_e1, W_e2, b_e2, W_m, b_m,
           W_h1, b_h1, W_h2, b_h2, W_x1, b_x1, W_x2):
    n = h.shape[0]
    e = edges.shape[1]
    epw = e // NW
    chunk = _pick_block(epw, 128)
    nchunk = epw // chunk

    i32 = edges[0].astype(jnp.int32)
    j32 = edges[1].astype(jnp.int32)

    w1a = W_e1[:C]
    w1b = W_e1[C:2 * C]
    wnp = W_e1[2 * C:2 * C + 2]

    ti, tj = _build_tables(h, w1a, w1b)
    xt = x.T
    gs, npx = _make_gather(e, n, chunk, nchunk)(ti, tj, xt, i32, j32)
    u, v4 = _edge_mlp(gs, npx, wnp, b_e1.reshape(1, C), W_e2,
                      b_e2.reshape(1, C), W_m, b_m.reshape(1, 1),
                      W_x1, b_x1.reshape(1, C), W_x2)
    z128 = jnp.zeros((n // NS, C), jnp.float32)
    z8 = jnp.zeros((n // NS, 8), jnp.float32)
    wm2, sg2 = _make_scatter(e, n, chunk, nchunk)(u, v4, i32, z128, z8)
    h_out, x_out = _node_update(
        h, x, wm2, sg2, W_h1[:C], W_h1[C:], b_h1.reshape(1, C),
        W_h2, b_h2.reshape(1, C))
    return (h_out, x_out)


# speedup vs baseline: 10.2897x; 1.0721x over previous
import jax
import jax.numpy as jnp
from jax import lax
from jax.experimental import pallas as pl
from jax.experimental.pallas import tpu as pltpu
from jax.experimental.pallas import tpu_sc as plsc

C = 128
TW = 144
NC = 2
NS = 16
NW = NC * NS
L = 16


def _psi(z):
    return jnp.sign(z) * jnp.log(jnp.abs(z) + 1.0)


def _pick_block(n, cap, mult=8):
    best = mult
    for b in range(mult, cap + 1, mult):
        if n % b == 0:
            best = b
    return best


def _tables_body(h_ref, w1a_ref, w1b_ref, ti_ref, tj_ref):
    hb = h_ref[...]
    ti_ref[...] = jnp.dot(hb, w1a_ref[...], preferred_element_type=jnp.float32)
    tj_ref[...] = jnp.dot(hb, w1b_ref[...], preferred_element_type=jnp.float32)


def _build_tables(h, w1a, w1b):
    n = h.shape[0]
    nb = _pick_block(n, 1024)
    grid = (n // nb,)
    return pl.pallas_call(
        _tables_body,
        grid=grid,
        in_specs=[
            pl.BlockSpec((nb, C), lambda i: (i, 0)),
            pl.BlockSpec((C, C), lambda i: (0, 0)),
            pl.BlockSpec((C, C), lambda i: (0, 0)),
        ],
        out_specs=[
            pl.BlockSpec((nb, C), lambda i: (i, 0)),
            pl.BlockSpec((nb, C), lambda i: (i, 0)),
        ],
        out_shape=[
            jax.ShapeDtypeStruct((n, C), jnp.float32),
            jax.ShapeDtypeStruct((n, C), jnp.float32),
        ],
    )(h, w1a, w1b)


def _make_gather(e, n, chunk, nchunk):
    epw = e // NW
    ngr = chunk // L
    last = nchunk - 1
    npairs = (nchunk + 1) // 2

    def body(ti_hbm, tj_hbm, x_hbm, i_hbm, j_hbm, gs_hbm, npx_hbm,
             ii0, jj0, a0, b0, o0, npx0, ii1, jj1, a1, b1, o1, npx1, x_loc,
             si0, sj0, sa0, sb0, sg0, sn0, si1, sj1, sa1, sb1, sg1, sn1):
        cc = lax.axis_index("c")
        ss = lax.axis_index("s")
        wid = ss * NC + cc
        base0 = wid * epw
        iota = lax.iota(jnp.int32, L)
        zv = jnp.zeros((L,), jnp.float32)
        bufs = [
            (ii0, jj0, a0, b0, o0, npx0, si0, sj0, sa0, sb0, sg0, sn0),
            (ii1, jj1, a1, b1, o1, npx1, si1, sj1, sa1, sb1, sg1, sn1),
        ]
        pltpu.sync_copy(x_hbm, x_loc)
        for _, _, _, _, _, npx_v, *_ in bufs:
            for g in range(ngr):
                npx_v[6, pl.ds(g * L, L)] = zv
                npx_v[7, pl.ds(g * L, L)] = zv

        def issue_idx(k, p):
            ii_v, jj_v = bufs[p][0], bufs[p][1]
            base = base0 + k * chunk
            pltpu.async_copy(i_hbm.at[pl.ds(base, chunk)], ii_v, bufs[p][6])
            pltpu.async_copy(j_hbm.at[pl.ds(base, chunk)], jj_v, bufs[p][7])

        def wait_idx(p):
            pltpu.make_async_copy(i_hbm.at[pl.ds(0, chunk)], bufs[p][0],
                                  bufs[p][6]).wait()
            pltpu.make_async_copy(j_hbm.at[pl.ds(0, chunk)], bufs[p][1],
                                  bufs[p][7]).wait()

        def issue_gather(p):
            pltpu.async_copy(ti_hbm.at[bufs[p][0]], bufs[p][2], bufs[p][8])
            pltpu.async_copy(tj_hbm.at[bufs[p][1]], bufs[p][3], bufs[p][9])

        def wait_gather(p):
            pltpu.make_async_copy(ti_hbm.at[bufs[p][0]], bufs[p][2],
                                  bufs[p][8]).wait()
            pltpu.make_async_copy(tj_hbm.at[bufs[p][1]], bufs[p][3],
                                  bufs[p][9]).wait()

        def issue_out(k, p):
            base = base0 + k * chunk
            pltpu.async_copy(bufs[p][4], gs_hbm.at[pl.ds(base, chunk)],
                             bufs[p][10])
            pltpu.async_copy(bufs[p][5], npx_hbm.at[:, pl.ds(base, chunk)],
                             bufs[p][11])

        def wait_out(p):
            pltpu.make_async_copy(bufs[p][4], gs_hbm.at[pl.ds(0, chunk)],
                                  bufs[p][10]).wait()
            pltpu.make_async_copy(bufs[p][5], npx_hbm.at[:, pl.ds(0, chunk)],
                                  bufs[p][11]).wait()

        def compute(p):
            ii_v, jj_v, a_v, b_v, o_v, npx_v, *_ = bufs[p]

            def row(ei, _):
                for k in range(C // L):
                    sl = pl.ds(k * L, L)
                    o_v[ei, sl] = a_v[ei, sl] + b_v[ei, sl]
                return 0

            lax.fori_loop(0, chunk, row, 0)
            for g in range(ngr):
                sl = pl.ds(g * L, L)
                i16 = ii_v[sl]
                j16 = jj_v[sl]
                xi = [plsc.load_gather(x_loc, [jnp.full((L,), c2, jnp.int32),
                                               i16])
                      for c2 in range(4)]
                xj = [plsc.load_gather(x_loc, [jnp.full((L,), c2, jnp.int32),
                                               j16])
                      for c2 in range(4)]
                d = [xi[c2] - xj[c2] for c2 in range(4)]
                nr = d[0] * d[0] - d[1] * d[1] - d[2] * d[2] - d[3] * d[3]
                pr = (xi[0] * xj[0] - xi[1] * xj[1] - xi[2] * xj[2]
                      - xi[3] * xj[3])
                npx_v[0, sl] = nr
                npx_v[1, sl] = pr
                for c2 in range(4):
                    npx_v[2 + c2, sl] = xj[c2]

        def handle(k, p):
            wait_gather(p)

            @pl.when(k + 1 <= last)
            def _():
                wait_idx(1 - p)
                issue_gather(1 - p)

            @pl.when(k >= 2)
            def _():
                wait_out(p)

            compute(p)

            @pl.when(k + 2 <= last)
            def _():
                issue_idx(k + 2, p)

            issue_out(k, p)

        issue_idx(0, 0)
        issue_idx(1, 1)
        wait_idx(0)
        issue_gather(0)

        def pair(m, _):
            handle(2 * m, 0)

            @pl.when(2 * m + 1 <= last)
            def _():
                handle(2 * m + 1, 1)

            return 0

        lax.fori_loop(0, npairs, pair, 0)
        wait_out(0)
        wait_out(1)

    mesh = plsc.VectorSubcoreMesh(core_axis_name="c", subcore_axis_name="s",
                                  num_cores=NC, num_subcores=NS)
    return pl.kernel(
        body,
        out_type=[
            jax.ShapeDtypeStruct((e, C), jnp.float32),
            jax.ShapeDtypeStruct((8, e), jnp.float32),
        ],
        mesh=mesh,
        compiler_params=pltpu.CompilerParams(use_tc_tiling_on_sc=False,
                                             needs_layout_passes=False),
        scratch_types=(
            [
                pltpu.VMEM((chunk,), jnp.int32),
                pltpu.VMEM((chunk,), jnp.int32),
                pltpu.VMEM((chunk, C), jnp.float32),
                pltpu.VMEM((chunk, C), jnp.float32),
                pltpu.VMEM((chunk, C), jnp.float32),
                pltpu.VMEM((8, chunk), jnp.float32),
            ] * 2
            + [pltpu.VMEM((4, n), jnp.float32)]
            + [pltpu.SemaphoreType.DMA] * 12
        ),
    )


def _edge_body(gs_ref, npx_ref, wnp_ref, be1_ref, we2_ref, be2_ref,
               wm_ref, bm_ref, wx1_ref, bx1_ref, wx2_ref, u_ref, v_ref):
    s = gs_ref[...]
    npx = npx_ref[...]
    nrp = _psi(npx[0:2, :])
    geo = lax.dot_general(nrp, wnp_ref[...], (((0,), (0,)), ((), ())),
                          preferred_element_type=jnp.float32)
    pre = s + geo + be1_ref[...]
    m1 = jnp.maximum(pre, 0.0)
    m2 = jnp.maximum(
        jnp.dot(m1, we2_ref[...], preferred_element_type=jnp.float32)
        + be2_ref[...], 0.0)
    att = jax.nn.sigmoid(
        jnp.dot(m2, wm_ref[...], preferred_element_type=jnp.float32)
        + bm_ref[...])
    u_ref[...] = att * m2
    t = jnp.maximum(
        jnp.dot(m2, wx1_ref[...], preferred_element_type=jnp.float32)
        + bx1_ref[...], 0.0)
    pxt = lax.dot_general(wx2_ref[...], t, (((0,), (1,)), ((), ())),
                          preferred_element_type=jnp.float32)
    v_ref[...] = pxt * npx[2:6, :]


def _edge_mlp(gs, npx, wnp, be1, we2, be2, wm, bm, wx1, bx1, wx2):
    e = gs.shape[0]
    be = _pick_block(e, 3200, mult=128)
    grid = (e // be,)
    full = lambda shp: pl.BlockSpec(shp, lambda i: (0,) * len(shp))
    return pl.pallas_call(
        _edge_body,
        grid=grid,
        in_specs=[
            pl.BlockSpec((be, C), lambda i: (i, 0)),
            pl.BlockSpec((8, be), lambda i: (0, i)),
            full((2, C)), full((1, C)),
            full((C, C)), full((1, C)),
            full((C, 1)), full((1, 1)),
            full((C, C)), full((1, C)), full((C, 1)),
        ],
        out_specs=[
            pl.BlockSpec((be, C), lambda i: (i, 0)),
            pl.BlockSpec((4, be), lambda i: (0, i)),
        ],
        out_shape=[
            jax.ShapeDtypeStruct((e, C), jnp.float32),
            jax.ShapeDtypeStruct((4, e), jnp.float32),
        ],
    )(gs, npx, wnp, be1, we2, be2, wm, bm, wx1, bx1, wx2)


def _make_scatter(e, n, chunk, nchunk):
    epw = e // NW
    rows_pt = n // NS
    ngr = chunk // L
    last = nchunk - 1
    ntrip = (nchunk + 2) // 3

    def body(u_hbm, v4_hbm, i_hbm, z128_hbm, z8_hbm, wm2_hbm, sg2_hbm,
             idx0, u0, v40, v0, idx1, u1, v41, v1, idx2, u2, v42, v2,
             wm_sh, sg_sh,
             li0, lu0, lv0, su0, sv0, li1, lu1, lv1, su1, sv1,
             li2, lu2, lv2, su2, sv2):
        cc = lax.axis_index("c")
        ss = lax.axis_index("s")
        wid = ss * NC + cc
        rowbase = ss * rows_pt
        iota = lax.iota(jnp.int32, L)
        bufs = [
            (idx0, u0, v40, v0, li0, lu0, lv0, su0, sv0),
            (idx1, u1, v41, v1, li1, lu1, lv1, su1, sv1),
            (idx2, u2, v42, v2, li2, lu2, lv2, su2, sv2),
        ]
        pltpu.sync_copy(z128_hbm, wm_sh.at[pl.ds(rowbase, rows_pt)])
        pltpu.sync_copy(z8_hbm, sg_sh.at[pl.ds(rowbase, rows_pt)])
        ones = jnp.ones((L,), jnp.float32)
        zv = jnp.zeros((L,), jnp.float32)
        for _, _, _, v_v, *_ in bufs:
            for g in range(ngr):
                rows = iota + g * L
                plsc.store_scatter(v_v, [rows, jnp.full((L,), 4, jnp.int32)],
                                   ones)
                for c2 in (5, 6, 7):
                    plsc.store_scatter(v_v,
                                       [rows, jnp.full((L,), c2, jnp.int32)],
                                       zv)
        plsc.subcore_barrier()

        def issue_loads(k, p):
            idx_v, u_v, v4_v, _, li, lu, lv, _, _ = bufs[p]
            base = wid * epw + k * chunk
            pltpu.async_copy(i_hbm.at[pl.ds(base, chunk)], idx_v, li)
            pltpu.async_copy(u_hbm.at[pl.ds(base, chunk)], u_v, lu)
            pltpu.async_copy(v4_hbm.at[:, pl.ds(base, chunk)], v4_v, lv)

        def wait_loads(p):
            idx_v, u_v, v4_v, _, li, lu, lv, _, _ = bufs[p]
            pltpu.make_async_copy(i_hbm.at[pl.ds(0, chunk)], idx_v, li).wait()
            pltpu.make_async_copy(u_hbm.at[pl.ds(0, chunk)], u_v, lu).wait()
            pltpu.make_async_copy(v4_hbm.at[:, pl.ds(0, chunk)], v4_v,
                                  lv).wait()

        def issue_scats(p):
            idx_v, u_v, _, v_v, _, _, _, su, sv = bufs[p]
            pltpu.async_copy(u_v, wm_sh.at[idx_v], su, add=True)
            pltpu.async_copy(v_v, sg_sh.at[idx_v], sv, add=True)

        def wait_scats(p):
            idx_v, u_v, _, v_v, _, _, _, su, sv = bufs[p]
            pltpu.make_async_copy(u_v, wm_sh.at[idx_v], su).wait()
            pltpu.make_async_copy(v_v, sg_sh.at[idx_v], sv).wait()

        def handle(k, p):
            nxt = (p + 1) % 3
            @pl.when(k + 1 <= last)
            def _():
                @pl.when(k >= 2)
                def _():
                    wait_scats(nxt)

                issue_loads(k + 1, nxt)

            wait_loads(p)
            _, _, v4_v, v_v, *_ = bufs[p]
            for g in range(ngr):
                rows = iota + g * L
                sl = pl.ds(g * L, L)
                for c2 in range(4):
                    plsc.store_scatter(
                        v_v, [rows, jnp.full((L,), c2, jnp.int32)],
                        v4_v[c2, sl])
            issue_scats(p)

        issue_loads(0, 0)

        def trip(m, _):
            handle(3 * m, 0)

            @pl.when(3 * m + 1 <= last)
            def _():
                handle(3 * m + 1, 1)

            @pl.when(3 * m + 2 <= last)
            def _():
                handle(3 * m + 2, 2)

            return 0

        lax.fori_loop(0, ntrip, trip, 0)
        wait_scats((last - 1) % 3)
        wait_scats(last % 3)
        plsc.subcore_barrier()
        pltpu.sync_copy(wm_sh.at[pl.ds(rowbase, rows_pt)],
                        wm2_hbm.at[cc, pl.ds(rowbase, rows_pt)])
        pltpu.sync_copy(sg_sh.at[pl.ds(rowbase, rows_pt)],
                        sg2_hbm.at[cc, pl.ds(rowbase, rows_pt)])

    mesh = plsc.VectorSubcoreMesh(core_axis_name="c", subcore_axis_name="s",
                                  num_cores=NC, num_subcores=NS)
    return pl.kernel(
        body,
        out_type=[
            jax.ShapeDtypeStruct((NC, n, C), jnp.float32),
            jax.ShapeDtypeStruct((NC, n, 8), jnp.float32),
        ],
        mesh=mesh,
        compiler_params=pltpu.CompilerParams(use_tc_tiling_on_sc=False,
                                             needs_layout_passes=False),
        scratch_types=(
            [
                pltpu.VMEM((chunk,), jnp.int32),
                pltpu.VMEM((chunk, C), jnp.float32),
                pltpu.VMEM((4, chunk), jnp.float32),
                pltpu.VMEM((chunk, 8), jnp.float32),
            ] * 3
            + [
                pltpu.VMEM_SHARED((n, C), jnp.float32),
                pltpu.VMEM_SHARED((n, 8), jnp.float32),
            ]
            + [pltpu.SemaphoreType.DMA] * 15
        ),
    )


def _node_body(h_ref, x_ref, wm2_ref, sg2_ref, wh1a_ref, wh1b_ref,
               bh1_ref, wh2_ref, bh2_ref, ho_ref, xo_ref):
    hb = h_ref[...]
    nparts = wm2_ref.shape[0]
    wm = wm2_ref[0]
    for q in range(1, nparts):
        wm = wm + wm2_ref[q]
    t = jnp.maximum(
        jnp.dot(hb, wh1a_ref[...], preferred_element_type=jnp.float32)
        + jnp.dot(wm, wh1b_ref[...], preferred_element_type=jnp.float32)
        + bh1_ref[...], 0.0)
    ho_ref[...] = (hb
                   + jnp.dot(t, wh2_ref[...],
                             preferred_element_type=jnp.float32)
                   + bh2_ref[...])
    sg = sg2_ref[0]
    for q in range(1, nparts):
        sg = sg + sg2_ref[q]
    cnt = jnp.maximum(sg[:, 4:5], 1.0)
    xo_ref[...] = x_ref[...] + 0.001 * (sg[:, 0:4] / cnt)


def _node_update(h, x, wm2, sg2, wh1a, wh1b, bh1, wh2, bh2):
    n = h.shape[0]
    nparts = wm2.shape[0]
    nb = _pick_block(n, 1024)
    grid = (n // nb,)
    full = lambda shp: pl.BlockSpec(shp, lambda i: (0,) * len(shp))
    return pl.pallas_call(
        _node_body,
        grid=grid,
        in_specs=[
            pl.BlockSpec((nb, C), lambda i: (i, 0)),
            pl.BlockSpec((nb, 4), lambda i: (i, 0)),
            pl.BlockSpec((nparts, nb, C), lambda i: (0, i, 0)),
            pl.BlockSpec((nparts, nb, 8), lambda i: (0, i, 0)),
            full((C, C)), full((C, C)), full((1, C)),
            full((C, C)), full((1, C)),
        ],
        out_specs=[
            pl.BlockSpec((nb, C), lambda i: (i, 0)),
            pl.BlockSpec((nb, 4), lambda i: (i, 0)),
        ],
        out_shape=[
            jax.ShapeDtypeStruct((n, C), jnp.float32),
            jax.ShapeDtypeStruct((n, 4), jnp.float32),
        ],
    )(h, x, wm2, sg2, wh1a, wh1b, bh1, wh2, bh2)


def kernel(x, h, edges, W_e1, b_e1, W_e2, b_e2, W_m, b_m,
           W_h1, b_h1, W_h2, b_h2, W_x1, b_x1, W_x2):
    n = h.shape[0]
    e = edges.shape[1]

    i32 = edges[0].astype(jnp.int32)
    j32 = edges[1].astype(jnp.int32)

    w1a = W_e1[:C]
    w1b = W_e1[C:2 * C]
    wnp = W_e1[2 * C:2 * C + 2]

    ti, tj = _build_tables(h, w1a, w1b)
    xt = x.T
    z128 = jnp.zeros((n // NS, C), jnp.float32)
    z8 = jnp.zeros((n // NS, 8), jnp.float32)

    grain = NW * 80
    e1 = (int(e * 0.55) // grain) * grain
    parts = [(0, e1), (e1, e)] if 0 < e1 < e else [(0, e)]

    uv = []
    for lo, hi in parts:
        ep = hi - lo
        epw = ep // NW
        chunk = _pick_block(epw, 128, mult=16)
        nchunk = epw // chunk
        ii, jj = i32[lo:hi], j32[lo:hi]
        gs, npx = _make_gather(ep, n, chunk, nchunk)(ti, tj, xt, ii, jj)
        u, v4 = _edge_mlp(gs, npx, wnp, b_e1.reshape(1, C), W_e2,
                          b_e2.reshape(1, C), W_m, b_m.reshape(1, 1),
                          W_x1, b_x1.reshape(1, C), W_x2)
        uv.append((ii, u, v4, ep, chunk, nchunk))

    wms, sgs = [], []
    for ii, u, v4, ep, chunk, nchunk in uv:
        wm2, sg2 = _make_scatter(ep, n, chunk, nchunk)(u, v4, ii, z128, z8)
        wms.append(wm2)
        sgs.append(sg2)
    wm_all = jnp.concatenate(wms, axis=0)
    sg_all = jnp.concatenate(sgs, axis=0)
    h_out, x_out = _node_update(
        h, x, wm_all, sg_all, W_h1[:C], W_h1[C:], b_h1.reshape(1, C),
        W_h2, b_h2.reshape(1, C))
    return (h_out, x_out)
